# Initial kernel scaffold; baseline (speedup 1.0000x reference)
import functools
import jax
import jax.numpy as jnp
from jax import lax
from jax.experimental import pallas as pl
from jax.experimental.pallas import tpu as pltpu
from jax.experimental.pallas import tpu_sc as plsc

N, E, D = 10000, 320000, 128
R, NB = 8, 30
NC, NS, L = 2, 16, 16
NW = NC * NS
EPW = E // NW
CH = 80
NCH = EPW // CH
NPT = N // NS
BLK = 400
NBLK = N // BLK

_mesh = plsc.VectorSubcoreMesh(
    core_axis_name="c", subcore_axis_name="s", num_cores=NC, num_subcores=NS)


def _proj_body(comp_ref, bases_ref, root_ref, x_ref, out_ref):
    r = pl.program_id(0)
    w = jnp.zeros((D, D), jnp.float32)
    for b in range(NB):
        w = w + comp_ref[r, b] * bases_ref[b]
    isroot = jnp.where(r == R, 1.0, 0.0)
    w = w + isroot * root_ref[...]
    out_ref[0] = jnp.dot(x_ref[...], w, preferred_element_type=jnp.float32)


def _proj(comp_pad, bases, root, x):
    return pl.pallas_call(
        _proj_body,
        grid=(R + 1, NBLK),
        in_specs=[
            pl.BlockSpec(memory_space=pltpu.SMEM),
            pl.BlockSpec((NB, D, D), lambda r, i: (0, 0, 0)),
            pl.BlockSpec((D, D), lambda r, i: (0, 0)),
            pl.BlockSpec((BLK, D), lambda r, i: (i, 0)),
        ],
        out_specs=pl.BlockSpec((1, BLK, D), lambda r, i: (r, i, 0)),
        out_shape=jax.ShapeDtypeStruct((R + 1, N, D), jnp.float32),
    )(comp_pad, bases, root, x)


def _zero_rows(rows):
    z = jnp.zeros((L,), jnp.float32)

    def body(j, _):
        def inner(k, __):
            rows[j, pl.ds(k * L, L)] = z
            return 0
        return lax.fori_loop(0, D // L, inner, 0)
    lax.fori_loop(0, CH, body, 0)


def _zero_spmem(rows, sh, s):
    for k in range(7):
        pltpu.sync_copy(rows, sh.at[pl.ds(s * NPT + k * CH, CH)])
    pltpu.sync_copy(rows.at[pl.ds(0, NPT - 7 * CH)],
                    sh.at[pl.ds(s * NPT + 7 * CH, NPT - 7 * CH)])


def _zero_1d(buf, n):
    z = jnp.zeros((L,), jnp.float32)

    def body(i, _):
        buf[pl.ds(i * L, L)] = z
        return 0
    lax.fori_loop(0, n // L, body, 0)


@functools.partial(
    pl.kernel,
    out_type=(jax.ShapeDtypeStruct((NC, N, D), jnp.float32),
              jax.ShapeDtypeStruct((NW, N), jnp.float32)),
    mesh=_mesh,
    scratch_types=[
        pltpu.VMEM((EPW,), jnp.int32),
        pltpu.VMEM((EPW,), jnp.int32),
        pltpu.VMEM((EPW,), jnp.int32),
        pltpu.VMEM((CH,), jnp.int32),
        pltpu.VMEM((CH,), jnp.int32),
        pltpu.VMEM((CH, D), jnp.float32),
        pltpu.VMEM((N,), jnp.float32),
        pltpu.VMEM_SHARED((N, D), jnp.float32),
        pltpu.SemaphoreType.DMA,
    ],
)
def _rgcn(table, src, dst, typ, agg_out, cnt_out,
          src_all, dst_all, typ_all, flatb, dstb, rows, cnt_local,
          agg_sh, gsem):
    c = lax.axis_index("c")
    s = lax.axis_index("s")
    wid = s * NC + c

    _zero_rows(rows)
    _zero_spmem(rows, agg_sh, s)
    _zero_1d(cnt_local, N)

    base = wid * EPW
    pltpu.sync_copy(src.at[pl.ds(base, EPW)], src_all)
    pltpu.sync_copy(dst.at[pl.ds(base, EPW)], dst_all)
    pltpu.sync_copy(typ.at[pl.ds(base, EPW)], typ_all)
    plsc.subcore_barrier()

    ones = jnp.ones((L,), jnp.float32)

    def chunk(ch, _):
        off = ch * CH
        for g in range(CH // L):
            sl = pl.ds(off + g * L, L)
            s16 = src_all[sl]
            t16 = typ_all[sl]
            d16 = dst_all[sl]
            flatb[pl.ds(g * L, L)] = t16 * N + s16
            dstb[pl.ds(g * L, L)] = d16
            plsc.addupdate_scatter(cnt_local, [d16], ones)
        pltpu.async_copy(table.at[flatb], rows, gsem).wait()
        pltpu.sync_copy(rows, agg_sh.at[dstb], add=True)
        return 0

    lax.fori_loop(0, NCH, chunk, 0)
    plsc.subcore_barrier()

    pltpu.sync_copy(agg_sh.at[pl.ds(s * NPT, NPT)],
                    agg_out.at[c, pl.ds(s * NPT, NPT)])
    pltpu.sync_copy(cnt_local, cnt_out.at[wid])


def _post_body(agg_ref, cnt_ref, xroot_ref, b1_ref, ll_ref, lr_ref,
               xl_ref, xr_ref, mm_ref):
    aggsum = agg_ref[0] + agg_ref[1]
    cnt = jnp.sum(cnt_ref[...], axis=0)
    x1 = (aggsum / jnp.clip(cnt, 1.0)[:, None]
          + xroot_ref[...] + b1_ref[...])
    xl = jnp.dot(x1, ll_ref[...], preferred_element_type=jnp.float32)
    xr = jnp.dot(x1, lr_ref[...], preferred_element_type=jnp.float32)
    xl_ref[...] = xl
    xr_ref[...] = xr
    mm_ref[0, 0] = jnp.max(xl, axis=0)
    mm_ref[0, 1] = jnp.min(xl, axis=0)


def _post(agg2, cnt32, xroot, bias1, lin_l, lin_r):
    return pl.pallas_call(
        _post_body,
        grid=(NBLK,),
        in_specs=[
            pl.BlockSpec((NC, BLK, D), lambda i: (0, i, 0)),
            pl.BlockSpec((NW, BLK), lambda i: (0, i)),
            pl.BlockSpec((BLK, D), lambda i: (i, 0)),
            pl.BlockSpec((1, D), lambda i: (0, 0)),
            pl.BlockSpec((D, D), lambda i: (0, 0)),
            pl.BlockSpec((D, D), lambda i: (0, 0)),
        ],
        out_specs=[
            pl.BlockSpec((BLK, D), lambda i: (i, 0)),
            pl.BlockSpec((BLK, D), lambda i: (i, 0)),
            pl.BlockSpec((1, 2, D), lambda i: (i, 0, 0)),
        ],
        out_shape=[
            jax.ShapeDtypeStruct((N, D), jnp.float32),
            jax.ShapeDtypeStruct((N, D), jnp.float32),
            jax.ShapeDtypeStruct((NBLK, 2, D), jnp.float32),
        ],
    )(agg2, cnt32, xroot, bias1, lin_l, lin_r)


def _bound_body(mm_ref, xr_ref, att_ref, mb_ref):
    xlmax = jnp.max(mm_ref[:, 0, :], axis=0)
    xlmin = jnp.min(mm_ref[:, 1, :], axis=0)
    attv = att_ref[0]
    ap = jnp.maximum(attv, 0.0)
    an = jnp.minimum(attv, 0.0)
    zp = xlmax[None, :] + xr_ref[...]
    zn = xlmin[None, :] + xr_ref[...]
    lrp = jnp.maximum(zp, 0.2 * zp)
    lrn = jnp.maximum(zn, 0.2 * zn)
    mb_ref[...] = jnp.sum(ap * lrp + an * lrn, axis=1, keepdims=True)


def _bound(mm, xr, att):
    return pl.pallas_call(
        _bound_body,
        grid=(NBLK,),
        in_specs=[
            pl.BlockSpec((NBLK, 2, D), lambda i: (0, 0, 0)),
            pl.BlockSpec((BLK, D), lambda i: (i, 0)),
            pl.BlockSpec((1, D), lambda i: (0, 0)),
        ],
        out_specs=pl.BlockSpec((BLK, 1), lambda i: (i, 0)),
        out_shape=jax.ShapeDtypeStruct((N, 1), jnp.float32),
    )(mm, xr, att)


@functools.partial(
    pl.kernel,
    out_type=(jax.ShapeDtypeStruct((NC, N, D), jnp.float32),
              jax.ShapeDtypeStruct((NW, N), jnp.float32)),
    mesh=_mesh,
    scratch_types=[
        pltpu.VMEM((EPW,), jnp.int32),
        pltpu.VMEM((EPW,), jnp.int32),
        pltpu.VMEM((CH,), jnp.int32),
        pltpu.VMEM((CH,), jnp.int32),
        pltpu.VMEM((CH, D), jnp.float32),
        pltpu.VMEM((CH, D), jnp.float32),
        pltpu.VMEM((N,), jnp.float32),
        pltpu.VMEM((N,), jnp.float32),
        pltpu.VMEM((D,), jnp.float32),
        pltpu.VMEM((L,), jnp.float32),
        pltpu.VMEM_SHARED((N, D), jnp.float32),
        pltpu.SemaphoreType.DMA,
        pltpu.SemaphoreType.DMA,
    ],
)
def _gat(xl, xr, mb, src, dst, att_in, num_out, den_out,
         src_all, dst_all, srcb, dstb, rows_l, rows_r, mb_local, den_local,
         att_v, pbuf, num_sh, lsem, rsem):
    c = lax.axis_index("c")
    s = lax.axis_index("s")
    wid = s * NC + c

    _zero_rows(rows_l)
    _zero_spmem(rows_l, num_sh, s)
    _zero_1d(den_local, N)

    base = wid * EPW
    pltpu.sync_copy(src.at[pl.ds(base, EPW)], src_all)
    pltpu.sync_copy(dst.at[pl.ds(base, EPW)], dst_all)
    pltpu.sync_copy(mb, mb_local)
    pltpu.sync_copy(att_in, att_v)
    plsc.subcore_barrier()

    iota = lax.iota(jnp.int32, L)

    def chunk(ch, _):
        off = ch * CH
        for g in range(CH // L):
            sl = pl.ds(off + g * L, L)
            srcb[pl.ds(g * L, L)] = src_all[sl]
            dstb[pl.ds(g * L, L)] = dst_all[sl]
        cl = pltpu.async_copy(xl.at[srcb], rows_l, lsem)
        cr = pltpu.async_copy(xr.at[dstb], rows_r, rsem)
        cl.wait()
        cr.wait()
        for g in range(CH // L):
            e16 = iota + (g * L)
            d16 = dstb[pl.ds(g * L, L)]

            def dstep(d, acc):
                dv = jnp.full((L,), d, jnp.int32)
                lv = plsc.load_gather(rows_l, [e16, dv])
                rv = plsc.load_gather(rows_r, [e16, dv])
                t = lv + rv
                lr = jnp.maximum(t, 0.2 * t)
                return acc + att_v[d] * lr

            score = lax.fori_loop(0, D, dstep, jnp.zeros((L,), jnp.float32))
            mb16 = plsc.load_gather(mb_local, [d16])
            p16 = jnp.exp(score - mb16)
            plsc.addupdate_scatter(den_local, [d16], p16)
            pbuf[...] = p16

            def scale(j, _):
                pj = pbuf[j]
                row = g * L + j
                for k in range(D // L):
                    ksl = pl.ds(k * L, L)
                    rows_l[row, ksl] = rows_l[row, ksl] * pj
                return 0

            lax.fori_loop(0, L, scale, 0)
        pltpu.sync_copy(rows_l, num_sh.at[dstb], add=True)
        return 0

    lax.fori_loop(0, NCH, chunk, 0)
    plsc.subcore_barrier()

    pltpu.sync_copy(num_sh.at[pl.ds(s * NPT, NPT)],
                    num_out.at[c, pl.ds(s * NPT, NPT)])
    pltpu.sync_copy(den_local, den_out.at[wid])


def _final_body(num_ref, den_ref, b2_ref, out_ref):
    nsum = num_ref[0] + num_ref[1]
    d = jnp.sum(den_ref[...], axis=0)[:, None]
    safe = jnp.where(d > 0.0, d, 1.0)
    out_ref[...] = jnp.where(d > 0.0, nsum / safe, 0.0) + b2_ref[...]


def _final(num2, den32, bias2):
    return pl.pallas_call(
        _final_body,
        grid=(NBLK,),
        in_specs=[
            pl.BlockSpec((NC, BLK, D), lambda i: (0, i, 0)),
            pl.BlockSpec((NW, BLK), lambda i: (0, i)),
            pl.BlockSpec((1, D), lambda i: (0, 0)),
        ],
        out_specs=pl.BlockSpec((BLK, D), lambda i: (i, 0)),
        out_shape=jax.ShapeDtypeStruct((N, D), jnp.float32),
    )(num2, den32, bias2)


def kernel(node_features, edge_index, edge_norm, edge_type, comp, bases,
           root, bias1, lin_l, lin_r, att, bias2):
    del edge_norm
    src = edge_index[0]
    dst = edge_index[1]

    comp_pad = jnp.concatenate([comp, jnp.zeros((1, NB), comp.dtype)], axis=0)
    xw_all = _proj(comp_pad, bases, root, node_features)
    table = xw_all[:R].reshape(R * N, D)
    xroot = xw_all[R]

    agg2, cnt32 = _rgcn(table, src, dst, edge_type)

    xl, xr, mm = _post(agg2, cnt32, xroot, bias1.reshape(1, D), lin_l, lin_r)
    mb = _bound(mm, xr, att.reshape(1, D))[:, 0]

    num2, den32 = _gat(xl, xr, mb, src, dst, att)

    return _final(num2, den32, bias2.reshape(1, D))


# speedup vs baseline: 4.0250x; 4.0250x over previous
import functools
import jax
import jax.numpy as jnp
from jax import lax
from jax.experimental import pallas as pl
from jax.experimental.pallas import tpu as pltpu
from jax.experimental.pallas import tpu_sc as plsc

N, E, D = 10000, 320000, 128
R, NB = 8, 30
NC, NS, L = 2, 16, 16
NW = NC * NS
EPW = E // NW
CH = 80
SUP = 2000
NSUP = EPW // SUP
SUBS = SUP // CH
SP0 = 624
BLK = 400
NBLK = N // BLK

_mesh = plsc.VectorSubcoreMesh(
    core_axis_name="c", subcore_axis_name="s", num_cores=NC, num_subcores=NS)


def _proj_body(comp_ref, bases_ref, root_ref, x_ref, out_ref):
    r = pl.program_id(0)
    w = jnp.zeros((D, D), jnp.float32)
    for b in range(NB):
        w = w + comp_ref[r, b] * bases_ref[b]
    isroot = jnp.where(r == R, 1.0, 0.0)
    w = w + isroot * root_ref[...]
    out_ref[0] = jnp.dot(x_ref[...], w, preferred_element_type=jnp.float32)


def _proj(comp_pad, bases, root, x):
    return pl.pallas_call(
        _proj_body,
        grid=(R + 1, NBLK),
        in_specs=[
            pl.BlockSpec(memory_space=pltpu.SMEM),
            pl.BlockSpec((NB, D, D), lambda r, i: (0, 0, 0)),
            pl.BlockSpec((D, D), lambda r, i: (0, 0)),
            pl.BlockSpec((BLK, D), lambda r, i: (i, 0)),
        ],
        out_specs=pl.BlockSpec((1, BLK, D), lambda r, i: (r, i, 0)),
        out_shape=jax.ShapeDtypeStruct((R + 1, N, D), jnp.float32),
    )(comp_pad, bases, root, x)


def _zero_rows(rows):
    z = jnp.zeros((L,), jnp.float32)

    def body(j, _):
        def inner(k, __):
            rows[j, pl.ds(k * L, L)] = z
            return 0
        return lax.fori_loop(0, D // L, inner, 0)
    lax.fori_loop(0, CH, body, 0)


def _zero_spmem(rows, sh, s):
    for k in range(7):
        pltpu.sync_copy(rows, sh.at[pl.ds(s * SP0 + k * CH, CH)])
    pltpu.sync_copy(rows.at[pl.ds(0, 64)], sh.at[pl.ds(s * SP0 + 7 * CH, 64)])

    @pl.when(s == NS - 1)
    def _():
        pltpu.sync_copy(rows.at[pl.ds(0, 16)], sh.at[pl.ds(N - 16, 16)])


def _zero_1d(buf, n):
    z = jnp.zeros((L,), jnp.float32)

    def body(i, _):
        buf[pl.ds(i * L, L)] = z
        return 0
    lax.fori_loop(0, n // L, body, 0)


@functools.partial(
    pl.kernel,
    out_type=(jax.ShapeDtypeStruct((NC, N, D), jnp.float32),
              jax.ShapeDtypeStruct((NW, 1, N), jnp.float32)),
    mesh=_mesh,
    compiler_params=pltpu.CompilerParams(needs_layout_passes=False),
    scratch_types=[
        pltpu.VMEM((SUP,), jnp.int32),
        pltpu.VMEM((SUP,), jnp.int32),
        pltpu.VMEM((SUP,), jnp.int32),
        pltpu.VMEM((CH,), jnp.int32),
        pltpu.VMEM((CH,), jnp.int32),
        pltpu.VMEM((CH, D), jnp.float32),
        pltpu.VMEM((N,), jnp.float32),
        pltpu.VMEM_SHARED((N, D), jnp.float32),
        pltpu.SemaphoreType.DMA,
    ],
)
def _rgcn(table, src, dst, typ, agg_out, cnt_out,
          src_sup, dst_sup, typ_sup, flatb, dstb, rows, cnt_local,
          agg_sh, gsem):
    c = lax.axis_index("c")
    s = lax.axis_index("s")
    wid = s * NC + c

    _zero_rows(rows)
    _zero_spmem(rows, agg_sh, s)
    _zero_1d(cnt_local, N)
    plsc.subcore_barrier()

    base = wid * EPW
    ones = jnp.ones((L,), jnp.float32)

    def sup_body(sc, _):
        sbase = base + sc * SUP
        pltpu.sync_copy(src.at[pl.ds(sbase, SUP)], src_sup)
        pltpu.sync_copy(dst.at[pl.ds(sbase, SUP)], dst_sup)
        pltpu.sync_copy(typ.at[pl.ds(sbase, SUP)], typ_sup)

        def chunk(cc, __):
            off = cc * CH
            for g in range(CH // L):
                sl = pl.ds(off + g * L, L)
                s16 = src_sup[sl]
                t16 = typ_sup[sl]
                d16 = dst_sup[sl]
                flatb[pl.ds(g * L, L)] = t16 * N + s16
                dstb[pl.ds(g * L, L)] = d16
                plsc.addupdate_scatter(cnt_local, [d16], ones)
            pltpu.async_copy(table.at[flatb], rows, gsem).wait()
            pltpu.sync_copy(rows, agg_sh.at[dstb], add=True)
            return 0

        return lax.fori_loop(0, SUBS, chunk, 0)

    lax.fori_loop(0, NSUP, sup_body, 0)
    plsc.subcore_barrier()

    pltpu.sync_copy(agg_sh.at[pl.ds(s * SP0, SP0)],
                    agg_out.at[c, pl.ds(s * SP0, SP0)])

    @pl.when(s == NS - 1)
    def _():
        pltpu.sync_copy(agg_sh.at[pl.ds(N - 16, 16)],
                        agg_out.at[c, pl.ds(N - 16, 16)])

    pltpu.sync_copy(cnt_local, cnt_out.at[wid, 0])


def _post_body(agg_ref, cnt_ref, xroot_ref, b1_ref, ll_ref, lr_ref,
               xl_ref, xr_ref, mm_ref):
    aggsum = agg_ref[0] + agg_ref[1]
    cnt = jnp.sum(cnt_ref[...], axis=1)
    x1 = (aggsum / jnp.clip(cnt, 1.0)[:, None]
          + xroot_ref[...] + b1_ref[...])
    xl = jnp.dot(x1, ll_ref[...], preferred_element_type=jnp.float32)
    xr = jnp.dot(x1, lr_ref[...], preferred_element_type=jnp.float32)
    xl_ref[...] = xl
    xr_ref[...] = xr
    mm_ref[0, 0] = jnp.max(xl, axis=0)
    mm_ref[0, 1] = jnp.min(xl, axis=0)


def _post(agg2, cnt32, xroot, bias1, lin_l, lin_r):
    return pl.pallas_call(
        _post_body,
        grid=(NBLK,),
        in_specs=[
            pl.BlockSpec((NC, BLK, D), lambda i: (0, i, 0)),
            pl.BlockSpec((BLK, NW), lambda i: (i, 0)),
            pl.BlockSpec((BLK, D), lambda i: (i, 0)),
            pl.BlockSpec((1, D), lambda i: (0, 0)),
            pl.BlockSpec((D, D), lambda i: (0, 0)),
            pl.BlockSpec((D, D), lambda i: (0, 0)),
        ],
        out_specs=[
            pl.BlockSpec((BLK, D), lambda i: (i, 0)),
            pl.BlockSpec((BLK, D), lambda i: (i, 0)),
            pl.BlockSpec((1, 2, D), lambda i: (i, 0, 0)),
        ],
        out_shape=[
            jax.ShapeDtypeStruct((N, D), jnp.float32),
            jax.ShapeDtypeStruct((N, D), jnp.float32),
            jax.ShapeDtypeStruct((NBLK, 2, D), jnp.float32),
        ],
    )(agg2, cnt32, xroot, bias1, lin_l, lin_r)


def _bound_body(mm_ref, xr_ref, att_ref, mb_ref):
    xlmax = jnp.max(mm_ref[:, 0, :], axis=0)
    xlmin = jnp.min(mm_ref[:, 1, :], axis=0)
    attv = att_ref[0]
    ap = jnp.maximum(attv, 0.0)
    an = jnp.minimum(attv, 0.0)
    zp = xlmax[None, :] + xr_ref[...]
    zn = xlmin[None, :] + xr_ref[...]
    lrp = jnp.maximum(zp, 0.2 * zp)
    lrn = jnp.maximum(zn, 0.2 * zn)
    mb_ref[...] = jnp.sum(ap * lrp + an * lrn, axis=1, keepdims=True)


def _bound(mm, xr, att):
    return pl.pallas_call(
        _bound_body,
        grid=(NBLK,),
        in_specs=[
            pl.BlockSpec((NBLK, 2, D), lambda i: (0, 0, 0)),
            pl.BlockSpec((BLK, D), lambda i: (i, 0)),
            pl.BlockSpec((1, D), lambda i: (0, 0)),
        ],
        out_specs=pl.BlockSpec((BLK, 1), lambda i: (i, 0)),
        out_shape=jax.ShapeDtypeStruct((N, 1), jnp.float32),
    )(mm, xr, att)


@functools.partial(
    pl.kernel,
    out_type=(jax.ShapeDtypeStruct((NC, N, D), jnp.float32),
              jax.ShapeDtypeStruct((NC, 1, N), jnp.float32)),
    mesh=_mesh,
    compiler_params=pltpu.CompilerParams(needs_layout_passes=False),
    scratch_types=[
        pltpu.VMEM((SUP,), jnp.int32),
        pltpu.VMEM((SUP,), jnp.int32),
        pltpu.VMEM((CH,), jnp.int32),
        pltpu.VMEM((CH,), jnp.int32),
        pltpu.VMEM((CH, D), jnp.float32),
        pltpu.VMEM((CH, D), jnp.float32),
        pltpu.VMEM((N,), jnp.float32),
        pltpu.VMEM((D,), jnp.float32),
        pltpu.VMEM((L,), jnp.float32),
        pltpu.VMEM_SHARED((N, D), jnp.float32),
        pltpu.VMEM_SHARED((N,), jnp.float32),
        pltpu.SemaphoreType.DMA,
        pltpu.SemaphoreType.DMA,
    ],
)
def _gat(xl, xr, mb, src, dst, att_in, num_out, den_out,
         src_sup, dst_sup, srcb, dstb, rows_l, rows_r, mb_local,
         att_v, pbuf, num_sh, den_sh, lsem, rsem):
    c = lax.axis_index("c")
    s = lax.axis_index("s")
    wid = s * NC + c

    _zero_rows(rows_l)
    _zero_spmem(rows_l, num_sh, s)
    _zero_1d(mb_local, N)

    @pl.when(s == 0)
    def _():
        pltpu.sync_copy(mb_local, den_sh)

    pltpu.sync_copy(mb, mb_local)
    pltpu.sync_copy(att_in, att_v)
    plsc.subcore_barrier()

    base = wid * EPW
    iota = lax.iota(jnp.int32, L)

    def sup_body(sc, _):
        sbase = base + sc * SUP
        pltpu.sync_copy(src.at[pl.ds(sbase, SUP)], src_sup)
        pltpu.sync_copy(dst.at[pl.ds(sbase, SUP)], dst_sup)

        def chunk(cc, __):
            off = cc * CH
            for g in range(CH // L):
                sl = pl.ds(off + g * L, L)
                srcb[pl.ds(g * L, L)] = src_sup[sl]
                dstb[pl.ds(g * L, L)] = dst_sup[sl]
            cl = pltpu.async_copy(xl.at[srcb], rows_l, lsem)
            cr = pltpu.async_copy(xr.at[dstb], rows_r, rsem)
            cl.wait()
            cr.wait()
            for g in range(CH // L):
                e16 = iota + (g * L)
                d16 = dstb[pl.ds(g * L, L)]

                def dstep(k, acc):
                    att16 = att_v[pl.ds(k * L, L)]

                    def sub(j, acc2):
                        dv = jnp.full((L,), k * L + j, jnp.int32)
                        lv = plsc.load_gather(rows_l, [e16, dv])
                        rv = plsc.load_gather(rows_r, [e16, dv])
                        t = lv + rv
                        lr = jnp.maximum(t, 0.2 * t)
                        return acc2 + att16[j] * lr

                    for j in range(L):
                        acc = sub(j, acc)
                    return acc

                score = lax.fori_loop(0, D // L, dstep,
                                      jnp.zeros((L,), jnp.float32))
                mb16 = plsc.load_gather(mb_local, [d16])
                p16 = jnp.exp(score - mb16)
                pbuf[...] = p16
                pltpu.sync_copy(pbuf, den_sh.at[d16], add=True)

                for j in range(L):
                    pj = p16[j]
                    row = g * L + j
                    for k in range(D // L):
                        ksl = pl.ds(k * L, L)
                        rows_l[row, ksl] = rows_l[row, ksl] * pj
            pltpu.sync_copy(rows_l, num_sh.at[dstb], add=True)
            return 0

        return lax.fori_loop(0, SUBS, chunk, 0)

    lax.fori_loop(0, NSUP, sup_body, 0)
    plsc.subcore_barrier()

    pltpu.sync_copy(num_sh.at[pl.ds(s * SP0, SP0)],
                    num_out.at[c, pl.ds(s * SP0, SP0)])

    @pl.when(s == NS - 1)
    def _():
        pltpu.sync_copy(num_sh.at[pl.ds(N - 16, 16)],
                        num_out.at[c, pl.ds(N - 16, 16)])

    @pl.when(s == 0)
    def _():
        pltpu.sync_copy(den_sh, den_out.at[c, 0])


def _final_body(num_ref, den_ref, b2_ref, out_ref):
    nsum = num_ref[0] + num_ref[1]
    d = jnp.sum(den_ref[...], axis=1)[:, None]
    safe = jnp.where(d > 0.0, d, 1.0)
    out_ref[...] = jnp.where(d > 0.0, nsum / safe, 0.0) + b2_ref[...]


def _final(num2, den32, bias2):
    return pl.pallas_call(
        _final_body,
        grid=(NBLK,),
        in_specs=[
            pl.BlockSpec((NC, BLK, D), lambda i: (0, i, 0)),
            pl.BlockSpec((BLK, NC), lambda i: (i, 0)),
            pl.BlockSpec((1, D), lambda i: (0, 0)),
        ],
        out_specs=pl.BlockSpec((BLK, D), lambda i: (i, 0)),
        out_shape=jax.ShapeDtypeStruct((N, D), jnp.float32),
    )(num2, den32, bias2)


def kernel(node_features, edge_index, edge_norm, edge_type, comp, bases,
           root, bias1, lin_l, lin_r, att, bias2):
    del edge_norm
    src = edge_index[0]
    dst = edge_index[1]

    comp_pad = jnp.concatenate([comp, jnp.zeros((1, NB), comp.dtype)], axis=0)
    xw_all = _proj(comp_pad, bases, root, node_features)
    table = xw_all[:R].reshape(R * N, D)
    xroot = xw_all[R]

    agg2, cnt32 = _rgcn(table, src, dst, edge_type)

    xl, xr, mm = _post(agg2, cnt32.reshape(NW, N).T, xroot,
                       bias1.reshape(1, D), lin_l, lin_r)
    mb = _bound(mm, xr, att.reshape(1, D))[:, 0]

    num2, den2 = _gat(xl, xr, mb, src, dst, att)

    return _final(num2, den2.reshape(NC, N).T, bias2.reshape(1, D))


# speedup vs baseline: 5.8963x; 1.4649x over previous
import functools
import jax
import jax.numpy as jnp
from jax import lax
from jax.experimental import pallas as pl
from jax.experimental.pallas import tpu as pltpu
from jax.experimental.pallas import tpu_sc as plsc

N, E, D = 10000, 320000, 128
R, NB = 8, 30
NC, NS, L = 2, 16, 16
NW = NC * NS
EPW = E // NW
CH = 80
SUP = 2000
NSUP = EPW // SUP
SUBS = SUP // CH
SP0 = 624
BLK = 400
NBLK = N // BLK

_mesh = plsc.VectorSubcoreMesh(
    core_axis_name="c", subcore_axis_name="s", num_cores=NC, num_subcores=NS)


def _proj_body(comp_ref, bases_ref, root_ref, x_ref, out_ref):
    r = pl.program_id(0)
    w = jnp.zeros((D, D), jnp.float32)
    for b in range(NB):
        w = w + comp_ref[r, b] * bases_ref[b]
    isroot = jnp.where(r == R, 1.0, 0.0)
    w = w + isroot * root_ref[...]
    out_ref[0] = jnp.dot(x_ref[...], w, preferred_element_type=jnp.float32)


def _proj(comp_pad, bases, root, x):
    return pl.pallas_call(
        _proj_body,
        grid=(R + 1, NBLK),
        in_specs=[
            pl.BlockSpec(memory_space=pltpu.SMEM),
            pl.BlockSpec((NB, D, D), lambda r, i: (0, 0, 0)),
            pl.BlockSpec((D, D), lambda r, i: (0, 0)),
            pl.BlockSpec((BLK, D), lambda r, i: (i, 0)),
        ],
        out_specs=pl.BlockSpec((1, BLK, D), lambda r, i: (r, i, 0)),
        out_shape=jax.ShapeDtypeStruct((R + 1, N, D), jnp.float32),
    )(comp_pad, bases, root, x)


def _zero_rows(rows):
    z = jnp.zeros((L,), jnp.float32)

    def body(j, _):
        def inner(k, __):
            rows[j, pl.ds(k * L, L)] = z
            return 0
        return lax.fori_loop(0, D // L, inner, 0)
    lax.fori_loop(0, CH, body, 0)


def _zero_spmem(rows, sh, s):
    for k in range(7):
        pltpu.sync_copy(rows, sh.at[pl.ds(s * SP0 + k * CH, CH)])
    pltpu.sync_copy(rows.at[pl.ds(0, 64)], sh.at[pl.ds(s * SP0 + 7 * CH, 64)])

    @pl.when(s == NS - 1)
    def _():
        pltpu.sync_copy(rows.at[pl.ds(0, 16)], sh.at[pl.ds(N - 16, 16)])


def _zero_1d(buf, n):
    z = jnp.zeros((L,), jnp.float32)

    def body(i, _):
        buf[pl.ds(i * L, L)] = z
        return 0
    lax.fori_loop(0, n // L, body, 0)


@functools.partial(
    pl.kernel,
    out_type=(jax.ShapeDtypeStruct((NC, N, D), jnp.float32),
              jax.ShapeDtypeStruct((NW, 1, N), jnp.float32)),
    mesh=_mesh,
    compiler_params=pltpu.CompilerParams(needs_layout_passes=False),
    scratch_types=[
        pltpu.VMEM((SUP,), jnp.int32),
        pltpu.VMEM((SUP,), jnp.int32),
        pltpu.VMEM((SUP,), jnp.int32),
        pltpu.VMEM((CH,), jnp.int32),
        pltpu.VMEM((CH,), jnp.int32),
        pltpu.VMEM((CH, D), jnp.float32),
        pltpu.VMEM((N,), jnp.float32),
        pltpu.VMEM_SHARED((N, D), jnp.float32),
        pltpu.SemaphoreType.DMA,
    ],
)
def _rgcn(table, src, dst, typ, agg_out, cnt_out,
          src_sup, dst_sup, typ_sup, flatb, dstb, rows, cnt_local,
          agg_sh, gsem):
    c = lax.axis_index("c")
    s = lax.axis_index("s")
    wid = s * NC + c

    _zero_rows(rows)
    _zero_spmem(rows, agg_sh, s)
    _zero_1d(cnt_local, N)
    plsc.subcore_barrier()

    base = wid * EPW
    ones = jnp.ones((L,), jnp.float32)

    def sup_body(sc, _):
        sbase = base + sc * SUP
        pltpu.sync_copy(src.at[pl.ds(sbase, SUP)], src_sup)
        pltpu.sync_copy(dst.at[pl.ds(sbase, SUP)], dst_sup)
        pltpu.sync_copy(typ.at[pl.ds(sbase, SUP)], typ_sup)

        def chunk(cc, __):
            off = cc * CH
            for g in range(CH // L):
                sl = pl.ds(off + g * L, L)
                s16 = src_sup[sl]
                t16 = typ_sup[sl]
                d16 = dst_sup[sl]
                flatb[pl.ds(g * L, L)] = t16 * N + s16
                dstb[pl.ds(g * L, L)] = d16
                plsc.addupdate_scatter(cnt_local, [d16], ones)
            pltpu.async_copy(table.at[flatb], rows, gsem).wait()
            pltpu.sync_copy(rows, agg_sh.at[dstb], add=True)
            return 0

        return lax.fori_loop(0, SUBS, chunk, 0)

    lax.fori_loop(0, NSUP, sup_body, 0)
    plsc.subcore_barrier()

    pltpu.sync_copy(agg_sh.at[pl.ds(s * SP0, SP0)],
                    agg_out.at[c, pl.ds(s * SP0, SP0)])

    @pl.when(s == NS - 1)
    def _():
        pltpu.sync_copy(agg_sh.at[pl.ds(N - 16, 16)],
                        agg_out.at[c, pl.ds(N - 16, 16)])

    pltpu.sync_copy(cnt_local, cnt_out.at[wid, 0])


def _post_body(agg_ref, cnt_ref, xroot_ref, b1_ref, ll_ref, lr_ref,
               xl_ref, xr_ref, mm_ref):
    aggsum = agg_ref[0] + agg_ref[1]
    cnt = jnp.sum(cnt_ref[...], axis=1)
    x1 = (aggsum / jnp.clip(cnt, 1.0)[:, None]
          + xroot_ref[...] + b1_ref[...])
    xl = jnp.dot(x1, ll_ref[...], preferred_element_type=jnp.float32)
    xr = jnp.dot(x1, lr_ref[...], preferred_element_type=jnp.float32)
    xl_ref[...] = xl
    xr_ref[...] = xr
    mm_ref[0, 0] = jnp.max(xl, axis=0)
    mm_ref[0, 1] = jnp.min(xl, axis=0)


def _post(agg2, cnt32, xroot, bias1, lin_l, lin_r):
    return pl.pallas_call(
        _post_body,
        grid=(NBLK,),
        in_specs=[
            pl.BlockSpec((NC, BLK, D), lambda i: (0, i, 0)),
            pl.BlockSpec((BLK, NW), lambda i: (i, 0)),
            pl.BlockSpec((BLK, D), lambda i: (i, 0)),
            pl.BlockSpec((1, D), lambda i: (0, 0)),
            pl.BlockSpec((D, D), lambda i: (0, 0)),
            pl.BlockSpec((D, D), lambda i: (0, 0)),
        ],
        out_specs=[
            pl.BlockSpec((BLK, D), lambda i: (i, 0)),
            pl.BlockSpec((BLK, D), lambda i: (i, 0)),
            pl.BlockSpec((1, 2, D), lambda i: (i, 0, 0)),
        ],
        out_shape=[
            jax.ShapeDtypeStruct((N, D), jnp.float32),
            jax.ShapeDtypeStruct((N, D), jnp.float32),
            jax.ShapeDtypeStruct((NBLK, 2, D), jnp.float32),
        ],
    )(agg2, cnt32, xroot, bias1, lin_l, lin_r)


def _bound_body(mm_ref, xr_ref, att_ref, mb_ref):
    xlmax = jnp.max(mm_ref[:, 0, :], axis=0)
    xlmin = jnp.min(mm_ref[:, 1, :], axis=0)
    attv = att_ref[0]
    ap = jnp.maximum(attv, 0.0)
    an = jnp.minimum(attv, 0.0)
    zp = xlmax[None, :] + xr_ref[...]
    zn = xlmin[None, :] + xr_ref[...]
    lrp = jnp.maximum(zp, 0.2 * zp)
    lrn = jnp.maximum(zn, 0.2 * zn)
    mb_ref[...] = jnp.sum(ap * lrp + an * lrn, axis=1, keepdims=True)


def _bound(mm, xr, att):
    return pl.pallas_call(
        _bound_body,
        grid=(NBLK,),
        in_specs=[
            pl.BlockSpec((NBLK, 2, D), lambda i: (0, 0, 0)),
            pl.BlockSpec((BLK, D), lambda i: (i, 0)),
            pl.BlockSpec((1, D), lambda i: (0, 0)),
        ],
        out_specs=pl.BlockSpec((BLK, 1), lambda i: (i, 0)),
        out_shape=jax.ShapeDtypeStruct((N, 1), jnp.float32),
    )(mm, xr, att)


@functools.partial(
    pl.kernel,
    out_type=(jax.ShapeDtypeStruct((E, D), jnp.float32),
              jax.ShapeDtypeStruct((E, D), jnp.float32),
              jax.ShapeDtypeStruct((E,), jnp.float32)),
    mesh=_mesh,
    compiler_params=pltpu.CompilerParams(needs_layout_passes=False),
    scratch_types=[
        pltpu.VMEM((SUP,), jnp.int32),
        pltpu.VMEM((SUP,), jnp.int32),
        pltpu.VMEM((CH,), jnp.int32),
        pltpu.VMEM((CH,), jnp.int32),
        pltpu.VMEM((CH, D), jnp.float32),
        pltpu.VMEM((CH, D), jnp.float32),
        pltpu.VMEM((CH,), jnp.float32),
        pltpu.VMEM((N,), jnp.float32),
        pltpu.SemaphoreType.DMA,
        pltpu.SemaphoreType.DMA,
    ],
)
def _egather(xl, xr, mb, src, dst, gxl_out, gxr_out, gmb_out,
             src_sup, dst_sup, srcb, dstb, rows_l, rows_r, gmbb,
             mb_local, lsem, rsem):
    c = lax.axis_index("c")
    s = lax.axis_index("s")
    wid = s * NC + c

    pltpu.sync_copy(mb, mb_local)

    base = wid * EPW

    def sup_body(sc, _):
        sbase = base + sc * SUP
        pltpu.sync_copy(src.at[pl.ds(sbase, SUP)], src_sup)
        pltpu.sync_copy(dst.at[pl.ds(sbase, SUP)], dst_sup)

        def chunk(cc, __):
            off = cc * CH
            for g in range(CH // L):
                sl = pl.ds(off + g * L, L)
                srcb[pl.ds(g * L, L)] = src_sup[sl]
                dstb[pl.ds(g * L, L)] = dst_sup[sl]
            cl = pltpu.async_copy(xl.at[srcb], rows_l, lsem)
            cr = pltpu.async_copy(xr.at[dstb], rows_r, rsem)
            for g in range(CH // L):
                d16 = dstb[pl.ds(g * L, L)]
                gmbb[pl.ds(g * L, L)] = plsc.load_gather(mb_local, [d16])
            cl.wait()
            cr.wait()
            ebase = sbase + off
            pltpu.sync_copy(rows_l, gxl_out.at[pl.ds(ebase, CH)])
            pltpu.sync_copy(rows_r, gxr_out.at[pl.ds(ebase, CH)])
            pltpu.sync_copy(gmbb, gmb_out.at[pl.ds(ebase, CH)])
            return 0

        return lax.fori_loop(0, SUBS, chunk, 0)

    lax.fori_loop(0, NSUP, sup_body, 0)


BE = 6400
NEB = E // BE


def _pw_body(gxl_ref, gxr_ref, gmb_ref, att_ref, pg_ref, p_ref):
    gxl = gxl_ref[...]
    t = gxl + gxr_ref[...]
    lr = jnp.maximum(t, 0.2 * t)
    score = jnp.dot(lr, att_ref[...], preferred_element_type=jnp.float32)
    p = jnp.exp(score - gmb_ref[...])
    pg_ref[...] = p * gxl
    p_ref[...] = p


def _pw(gxl, gxr, gmb2, att2):
    return pl.pallas_call(
        _pw_body,
        grid=(NEB,),
        in_specs=[
            pl.BlockSpec((BE, D), lambda i: (i, 0)),
            pl.BlockSpec((BE, D), lambda i: (i, 0)),
            pl.BlockSpec((BE, 1), lambda i: (i, 0)),
            pl.BlockSpec((D, 1), lambda i: (0, 0)),
        ],
        out_specs=[
            pl.BlockSpec((BE, D), lambda i: (i, 0)),
            pl.BlockSpec((BE, 1), lambda i: (i, 0)),
        ],
        out_shape=[
            jax.ShapeDtypeStruct((E, D), jnp.float32),
            jax.ShapeDtypeStruct((E, 1), jnp.float32),
        ],
    )(gxl, gxr, gmb2, att2)


@functools.partial(
    pl.kernel,
    out_type=(jax.ShapeDtypeStruct((NC, N, D), jnp.float32),
              jax.ShapeDtypeStruct((NC, 1, N), jnp.float32)),
    mesh=_mesh,
    compiler_params=pltpu.CompilerParams(needs_layout_passes=False),
    scratch_types=[
        pltpu.VMEM((SUP,), jnp.int32),
        pltpu.VMEM((CH,), jnp.int32),
        pltpu.VMEM((CH, D), jnp.float32),
        pltpu.VMEM((CH,), jnp.float32),
        pltpu.VMEM((N,), jnp.float32),
        pltpu.VMEM_SHARED((N, D), jnp.float32),
        pltpu.VMEM_SHARED((N,), jnp.float32),
        pltpu.SemaphoreType.DMA,
    ],
)
def _escatter(pg, pvec, dst, num_out, den_out,
              dst_sup, dstb, rows, pb, zbuf, num_sh, den_sh, gsem):
    c = lax.axis_index("c")
    s = lax.axis_index("s")
    wid = s * NC + c

    _zero_rows(rows)
    _zero_spmem(rows, num_sh, s)
    _zero_1d(zbuf, N)

    @pl.when(s == 0)
    def _():
        pltpu.sync_copy(zbuf, den_sh)

    plsc.subcore_barrier()

    base = wid * EPW

    def sup_body(sc, _):
        sbase = base + sc * SUP
        pltpu.sync_copy(dst.at[pl.ds(sbase, SUP)], dst_sup)

        def chunk(cc, __):
            off = cc * CH
            for g in range(CH // L):
                dstb[pl.ds(g * L, L)] = dst_sup[pl.ds(off + g * L, L)]
            ebase = sbase + off
            cg = pltpu.async_copy(pg.at[pl.ds(ebase, CH)], rows, gsem)
            pltpu.sync_copy(pvec.at[pl.ds(ebase, CH)], pb)
            cg.wait()
            pltpu.sync_copy(pb, den_sh.at[dstb], add=True)
            pltpu.sync_copy(rows, num_sh.at[dstb], add=True)
            return 0

        return lax.fori_loop(0, SUBS, chunk, 0)

    lax.fori_loop(0, NSUP, sup_body, 0)
    plsc.subcore_barrier()

    pltpu.sync_copy(num_sh.at[pl.ds(s * SP0, SP0)],
                    num_out.at[c, pl.ds(s * SP0, SP0)])

    @pl.when(s == NS - 1)
    def _():
        pltpu.sync_copy(num_sh.at[pl.ds(N - 16, 16)],
                        num_out.at[c, pl.ds(N - 16, 16)])

    @pl.when(s == 0)
    def _():
        pltpu.sync_copy(den_sh, den_out.at[c, 0])


def _final_body(num_ref, den_ref, b2_ref, out_ref):
    nsum = num_ref[0] + num_ref[1]
    d = jnp.sum(den_ref[...], axis=1)[:, None]
    safe = jnp.where(d > 0.0, d, 1.0)
    out_ref[...] = jnp.where(d > 0.0, nsum / safe, 0.0) + b2_ref[...]


def _final(num2, den32, bias2):
    return pl.pallas_call(
        _final_body,
        grid=(NBLK,),
        in_specs=[
            pl.BlockSpec((NC, BLK, D), lambda i: (0, i, 0)),
            pl.BlockSpec((BLK, NC), lambda i: (i, 0)),
            pl.BlockSpec((1, D), lambda i: (0, 0)),
        ],
        out_specs=pl.BlockSpec((BLK, D), lambda i: (i, 0)),
        out_shape=jax.ShapeDtypeStruct((N, D), jnp.float32),
    )(num2, den32, bias2)


def kernel(node_features, edge_index, edge_norm, edge_type, comp, bases,
           root, bias1, lin_l, lin_r, att, bias2):
    del edge_norm
    src = edge_index[0]
    dst = edge_index[1]

    comp_pad = jnp.concatenate([comp, jnp.zeros((1, NB), comp.dtype)], axis=0)
    xw_all = _proj(comp_pad, bases, root, node_features)
    table = xw_all[:R].reshape(R * N, D)
    xroot = xw_all[R]

    agg2, cnt32 = _rgcn(table, src, dst, edge_type)

    xl, xr, mm = _post(agg2, cnt32.reshape(NW, N).T, xroot,
                       bias1.reshape(1, D), lin_l, lin_r)
    mb = _bound(mm, xr, att.reshape(1, D))[:, 0]

    gxl, gxr, gmb = _egather(xl, xr, mb, src, dst)
    pg, p2 = _pw(gxl, gxr, gmb.reshape(E, 1), att.reshape(D, 1))
    num2, den2 = _escatter(pg, p2.reshape(E), dst)

    return _final(num2, den2.reshape(NC, N).T, bias2.reshape(1, D))


# speedup vs baseline: 6.5008x; 1.1025x over previous
import functools
import jax
import jax.numpy as jnp
from jax import lax
from jax.experimental import pallas as pl
from jax.experimental.pallas import tpu as pltpu
from jax.experimental.pallas import tpu_sc as plsc

N, E, D = 10000, 320000, 128
R, NB = 8, 30
NC, NS, L = 2, 16, 16
NW = NC * NS
EPW = E // NW
CH = 80
SUP = 2000
NSUP = EPW // SUP
SUBS = SUP // CH
SP0 = 624
BLK = 400
NBLK = N // BLK

_mesh = plsc.VectorSubcoreMesh(
    core_axis_name="c", subcore_axis_name="s", num_cores=NC, num_subcores=NS)


def _proj_body(comp_ref, bases_ref, root_ref, x_ref, out_ref):
    r = pl.program_id(0)
    w = jnp.zeros((D, D), jnp.float32)
    for b in range(NB):
        w = w + comp_ref[r, b] * bases_ref[b]
    isroot = jnp.where(r == R, 1.0, 0.0)
    w = w + isroot * root_ref[...]
    out_ref[0] = jnp.dot(x_ref[...], w, preferred_element_type=jnp.float32)


def _proj(comp_pad, bases, root, x):
    return pl.pallas_call(
        _proj_body,
        grid=(R + 1, NBLK),
        in_specs=[
            pl.BlockSpec(memory_space=pltpu.SMEM),
            pl.BlockSpec((NB, D, D), lambda r, i: (0, 0, 0)),
            pl.BlockSpec((D, D), lambda r, i: (0, 0)),
            pl.BlockSpec((BLK, D), lambda r, i: (i, 0)),
        ],
        out_specs=pl.BlockSpec((1, BLK, D), lambda r, i: (r, i, 0)),
        out_shape=jax.ShapeDtypeStruct((R + 1, N, D), jnp.float32),
    )(comp_pad, bases, root, x)


def _zero_rows(rows):
    z = jnp.zeros((L,), jnp.float32)

    def body(j, _):
        def inner(k, __):
            rows[j, pl.ds(k * L, L)] = z
            return 0
        return lax.fori_loop(0, D // L, inner, 0)
    lax.fori_loop(0, CH, body, 0)


def _zero_spmem(rows, sh, s):
    for k in range(7):
        pltpu.sync_copy(rows, sh.at[pl.ds(s * SP0 + k * CH, CH)])
    pltpu.sync_copy(rows.at[pl.ds(0, 64)], sh.at[pl.ds(s * SP0 + 7 * CH, 64)])

    @pl.when(s == NS - 1)
    def _():
        pltpu.sync_copy(rows.at[pl.ds(0, 16)], sh.at[pl.ds(N - 16, 16)])


def _zero_1d(buf, n):
    z = jnp.zeros((L,), jnp.float32)

    def body(i, _):
        buf[pl.ds(i * L, L)] = z
        return 0
    lax.fori_loop(0, n // L, body, 0)


@functools.partial(
    pl.kernel,
    out_type=(jax.ShapeDtypeStruct((NC, N, D), jnp.float32),
              jax.ShapeDtypeStruct((NW, 1, N), jnp.float32)),
    mesh=_mesh,
    compiler_params=pltpu.CompilerParams(needs_layout_passes=False),
    scratch_types=[
        pltpu.VMEM((SUP,), jnp.int32),
        pltpu.VMEM((SUP,), jnp.int32),
        pltpu.VMEM((SUP,), jnp.int32),
        pltpu.VMEM((CH,), jnp.int32),
        pltpu.VMEM((CH,), jnp.int32),
        pltpu.VMEM((CH, D), jnp.float32),
        pltpu.VMEM((N,), jnp.float32),
        pltpu.VMEM_SHARED((N, D), jnp.float32),
        pltpu.SemaphoreType.DMA,
    ],
)
def _rgcn(table, src, dst, typ, agg_out, cnt_out,
          src_sup, dst_sup, typ_sup, flatb, dstb, rows, cnt_local,
          agg_sh, gsem):
    c = lax.axis_index("c")
    s = lax.axis_index("s")
    wid = s * NC + c

    _zero_rows(rows)
    _zero_spmem(rows, agg_sh, s)
    _zero_1d(cnt_local, N)
    plsc.subcore_barrier()

    base = wid * EPW
    ones = jnp.ones((L,), jnp.float32)

    def sup_body(sc, _):
        sbase = base + sc * SUP
        pltpu.sync_copy(src.at[pl.ds(sbase, SUP)], src_sup)
        pltpu.sync_copy(dst.at[pl.ds(sbase, SUP)], dst_sup)
        pltpu.sync_copy(typ.at[pl.ds(sbase, SUP)], typ_sup)

        def chunk(cc, __):
            off = cc * CH
            for g in range(CH // L):
                sl = pl.ds(off + g * L, L)
                s16 = src_sup[sl]
                t16 = typ_sup[sl]
                d16 = dst_sup[sl]
                flatb[pl.ds(g * L, L)] = t16 * N + s16
                dstb[pl.ds(g * L, L)] = d16
                plsc.addupdate_scatter(cnt_local, [d16], ones)
            pltpu.async_copy(table.at[flatb], rows, gsem).wait()
            pltpu.sync_copy(rows, agg_sh.at[dstb], add=True)
            return 0

        return lax.fori_loop(0, SUBS, chunk, 0)

    lax.fori_loop(0, NSUP, sup_body, 0)
    plsc.subcore_barrier()

    pltpu.sync_copy(agg_sh.at[pl.ds(s * SP0, SP0)],
                    agg_out.at[c, pl.ds(s * SP0, SP0)])

    @pl.when(s == NS - 1)
    def _():
        pltpu.sync_copy(agg_sh.at[pl.ds(N - 16, 16)],
                        agg_out.at[c, pl.ds(N - 16, 16)])

    pltpu.sync_copy(cnt_local, cnt_out.at[wid, 0])


def _post_body(agg_ref, cnt_ref, xroot_ref, b1_ref, ll_ref, lr_ref,
               xl_ref, xr_ref, mm_ref):
    aggsum = agg_ref[0] + agg_ref[1]
    cnt = jnp.sum(cnt_ref[...], axis=1)
    x1 = (aggsum / jnp.clip(cnt, 1.0)[:, None]
          + xroot_ref[...] + b1_ref[...])
    xl = jnp.dot(x1, ll_ref[...], preferred_element_type=jnp.float32)
    xr = jnp.dot(x1, lr_ref[...], preferred_element_type=jnp.float32)
    xl_ref[...] = xl
    xr_ref[...] = xr
    mm_ref[0, 0] = jnp.max(xl, axis=0)
    mm_ref[0, 1] = jnp.min(xl, axis=0)


def _post(agg2, cnt32, xroot, bias1, lin_l, lin_r):
    return pl.pallas_call(
        _post_body,
        grid=(NBLK,),
        in_specs=[
            pl.BlockSpec((NC, BLK, D), lambda i: (0, i, 0)),
            pl.BlockSpec((BLK, NW), lambda i: (i, 0)),
            pl.BlockSpec((BLK, D), lambda i: (i, 0)),
            pl.BlockSpec((1, D), lambda i: (0, 0)),
            pl.BlockSpec((D, D), lambda i: (0, 0)),
            pl.BlockSpec((D, D), lambda i: (0, 0)),
        ],
        out_specs=[
            pl.BlockSpec((BLK, D), lambda i: (i, 0)),
            pl.BlockSpec((BLK, D), lambda i: (i, 0)),
            pl.BlockSpec((1, 2, D), lambda i: (i, 0, 0)),
        ],
        out_shape=[
            jax.ShapeDtypeStruct((N, D), jnp.float32),
            jax.ShapeDtypeStruct((N, D), jnp.float32),
            jax.ShapeDtypeStruct((NBLK, 2, D), jnp.float32),
        ],
    )(agg2, cnt32, xroot, bias1, lin_l, lin_r)


def _bound_body(mm_ref, xr_ref, att_ref, mb_ref):
    xlmax = jnp.max(mm_ref[:, 0, :], axis=0)
    xlmin = jnp.min(mm_ref[:, 1, :], axis=0)
    attv = att_ref[0]
    ap = jnp.maximum(attv, 0.0)
    an = jnp.minimum(attv, 0.0)
    zp = xlmax[None, :] + xr_ref[...]
    zn = xlmin[None, :] + xr_ref[...]
    lrp = jnp.maximum(zp, 0.2 * zp)
    lrn = jnp.maximum(zn, 0.2 * zn)
    mb_ref[...] = jnp.sum(ap * lrp + an * lrn, axis=1, keepdims=True)


def _bound(mm, xr, att):
    return pl.pallas_call(
        _bound_body,
        grid=(NBLK,),
        in_specs=[
            pl.BlockSpec((NBLK, 2, D), lambda i: (0, 0, 0)),
            pl.BlockSpec((BLK, D), lambda i: (i, 0)),
            pl.BlockSpec((1, D), lambda i: (0, 0)),
        ],
        out_specs=pl.BlockSpec((BLK, 1), lambda i: (i, 0)),
        out_shape=jax.ShapeDtypeStruct((N, 1), jnp.float32),
    )(mm, xr, att)


def _make_egather(eh):
    epw = eh // NW
    nsup = epw // SUP

    @functools.partial(
        pl.kernel,
        out_type=(jax.ShapeDtypeStruct((eh, D), jnp.float32),
                  jax.ShapeDtypeStruct((eh, D), jnp.float32),
                  jax.ShapeDtypeStruct((eh,), jnp.float32)),
        mesh=_mesh,
        compiler_params=pltpu.CompilerParams(needs_layout_passes=False),
        scratch_types=[
            pltpu.VMEM((SUP,), jnp.int32),
            pltpu.VMEM((SUP,), jnp.int32),
            pltpu.VMEM((CH,), jnp.int32),
            pltpu.VMEM((CH,), jnp.int32),
            pltpu.VMEM((CH, D), jnp.float32),
            pltpu.VMEM((CH, D), jnp.float32),
            pltpu.VMEM((CH,), jnp.float32),
            pltpu.VMEM((N,), jnp.float32),
            pltpu.SemaphoreType.DMA,
            pltpu.SemaphoreType.DMA,
        ],
    )
    def _eg(xl, xr, mb, src, dst, gxl_out, gxr_out, gmb_out,
            src_sup, dst_sup, srcb, dstb, rows_l, rows_r, gmbb,
            mb_local, lsem, rsem):
        c = lax.axis_index("c")
        s = lax.axis_index("s")
        wid = s * NC + c

        pltpu.sync_copy(mb, mb_local)

        base = wid * epw

        def sup_body(sc, _):
            sbase = base + sc * SUP
            pltpu.sync_copy(src.at[pl.ds(sbase, SUP)], src_sup)
            pltpu.sync_copy(dst.at[pl.ds(sbase, SUP)], dst_sup)

            def chunk(cc, __):
                off = cc * CH
                for g in range(CH // L):
                    sl = pl.ds(off + g * L, L)
                    srcb[pl.ds(g * L, L)] = src_sup[sl]
                    dstb[pl.ds(g * L, L)] = dst_sup[sl]
                cl = pltpu.async_copy(xl.at[srcb], rows_l, lsem)
                cr = pltpu.async_copy(xr.at[dstb], rows_r, rsem)
                for g in range(CH // L):
                    d16 = dstb[pl.ds(g * L, L)]
                    gmbb[pl.ds(g * L, L)] = plsc.load_gather(mb_local, [d16])
                cl.wait()
                cr.wait()
                ebase = sbase + off
                pltpu.sync_copy(rows_l, gxl_out.at[pl.ds(ebase, CH)])
                pltpu.sync_copy(rows_r, gxr_out.at[pl.ds(ebase, CH)])
                pltpu.sync_copy(gmbb, gmb_out.at[pl.ds(ebase, CH)])
                return 0

            return lax.fori_loop(0, SUBS, chunk, 0)

        lax.fori_loop(0, nsup, sup_body, 0)

    return _eg


BE = 6400


def _pw_body(gxl_ref, gxr_ref, gmb_ref, att_ref, pg_ref, p_ref):
    gxl = gxl_ref[...]
    t = gxl + gxr_ref[...]
    lr = jnp.maximum(t, 0.2 * t)
    score = jnp.dot(lr, att_ref[...], preferred_element_type=jnp.float32)
    p = jnp.exp(score - gmb_ref[...])
    pg_ref[...] = p * gxl
    p_ref[...] = p


def _pw(gxl, gxr, gmb2, att2):
    eh = gxl.shape[0]
    return pl.pallas_call(
        _pw_body,
        grid=(eh // BE,),
        in_specs=[
            pl.BlockSpec((BE, D), lambda i: (i, 0)),
            pl.BlockSpec((BE, D), lambda i: (i, 0)),
            pl.BlockSpec((BE, 1), lambda i: (i, 0)),
            pl.BlockSpec((D, 1), lambda i: (0, 0)),
        ],
        out_specs=[
            pl.BlockSpec((BE, D), lambda i: (i, 0)),
            pl.BlockSpec((BE, 1), lambda i: (i, 0)),
        ],
        out_shape=[
            jax.ShapeDtypeStruct((eh, D), jnp.float32),
            jax.ShapeDtypeStruct((eh, 1), jnp.float32),
        ],
    )(gxl, gxr, gmb2, att2)


def _make_escatter(eh):
    epw = eh // NW
    nsup = epw // SUP

    @functools.partial(
        pl.kernel,
        out_type=(jax.ShapeDtypeStruct((NC, N, D), jnp.float32),
                  jax.ShapeDtypeStruct((NC, 1, N), jnp.float32)),
        mesh=_mesh,
        compiler_params=pltpu.CompilerParams(needs_layout_passes=False),
        scratch_types=[
            pltpu.VMEM((SUP,), jnp.int32),
            pltpu.VMEM((CH,), jnp.int32),
            pltpu.VMEM((CH, D), jnp.float32),
            pltpu.VMEM((CH,), jnp.float32),
            pltpu.VMEM((N,), jnp.float32),
            pltpu.VMEM_SHARED((N, D), jnp.float32),
            pltpu.VMEM_SHARED((N,), jnp.float32),
            pltpu.SemaphoreType.DMA,
        ],
    )
    def _es(pg, pvec, dst, num_out, den_out,
            dst_sup, dstb, rows, pb, zbuf, num_sh, den_sh, gsem):
        c = lax.axis_index("c")
        s = lax.axis_index("s")
        wid = s * NC + c

        _zero_rows(rows)
        _zero_spmem(rows, num_sh, s)
        _zero_1d(zbuf, N)

        @pl.when(s == 0)
        def _():
            pltpu.sync_copy(zbuf, den_sh)

        plsc.subcore_barrier()

        base = wid * epw

        def sup_body(sc, _):
            sbase = base + sc * SUP
            pltpu.sync_copy(dst.at[pl.ds(sbase, SUP)], dst_sup)

            def chunk(cc, __):
                off = cc * CH
                for g in range(CH // L):
                    dstb[pl.ds(g * L, L)] = dst_sup[pl.ds(off + g * L, L)]
                ebase = sbase + off
                cg = pltpu.async_copy(pg.at[pl.ds(ebase, CH)], rows, gsem)
                pltpu.sync_copy(pvec.at[pl.ds(ebase, CH)], pb)
                cg.wait()
                pltpu.sync_copy(pb, den_sh.at[dstb], add=True)
                pltpu.sync_copy(rows, num_sh.at[dstb], add=True)
                return 0

            return lax.fori_loop(0, SUBS, chunk, 0)

        lax.fori_loop(0, nsup, sup_body, 0)
        plsc.subcore_barrier()

        pltpu.sync_copy(num_sh.at[pl.ds(s * SP0, SP0)],
                        num_out.at[c, pl.ds(s * SP0, SP0)])

        @pl.when(s == NS - 1)
        def _():
            pltpu.sync_copy(num_sh.at[pl.ds(N - 16, 16)],
                            num_out.at[c, pl.ds(N - 16, 16)])

        @pl.when(s == 0)
        def _():
            pltpu.sync_copy(den_sh, den_out.at[c, 0])

    return _es


def _final_body(num0_ref, num1_ref, den_ref, b2_ref, out_ref):
    nsum = num0_ref[0] + num0_ref[1] + num1_ref[0] + num1_ref[1]
    d = jnp.sum(den_ref[...], axis=1)[:, None]
    safe = jnp.where(d > 0.0, d, 1.0)
    out_ref[...] = jnp.where(d > 0.0, nsum / safe, 0.0) + b2_ref[...]


def _final(num0, num1, den4, bias2):
    return pl.pallas_call(
        _final_body,
        grid=(NBLK,),
        in_specs=[
            pl.BlockSpec((NC, BLK, D), lambda i: (0, i, 0)),
            pl.BlockSpec((NC, BLK, D), lambda i: (0, i, 0)),
            pl.BlockSpec((BLK, 2 * NC), lambda i: (i, 0)),
            pl.BlockSpec((1, D), lambda i: (0, 0)),
        ],
        out_specs=pl.BlockSpec((BLK, D), lambda i: (i, 0)),
        out_shape=jax.ShapeDtypeStruct((N, D), jnp.float32),
    )(num0, num1, den4, bias2)


E0 = 192000
E1 = E - E0
_eg0 = _make_egather(E0)
_eg1 = _make_egather(E1)
_es0 = _make_escatter(E0)
_es1 = _make_escatter(E1)


def kernel(node_features, edge_index, edge_norm, edge_type, comp, bases,
           root, bias1, lin_l, lin_r, att, bias2):
    del edge_norm
    src = edge_index[0]
    dst = edge_index[1]

    comp_pad = jnp.concatenate([comp, jnp.zeros((1, NB), comp.dtype)], axis=0)
    xw_all = _proj(comp_pad, bases, root, node_features)
    table = xw_all[:R].reshape(R * N, D)
    xroot = xw_all[R]

    agg2, cnt32 = _rgcn(table, src, dst, edge_type)

    xl, xr, mm = _post(agg2, cnt32.reshape(NW, N).T, xroot,
                       bias1.reshape(1, D), lin_l, lin_r)
    mb = _bound(mm, xr, att.reshape(1, D))[:, 0]

    src0, dst0 = src[:E0], dst[:E0]
    src1, dst1 = src[E0:], dst[E0:]
    att2 = att.reshape(D, 1)

    gxl0, gxr0, gmb0 = _eg0(xl, xr, mb, src0, dst0)
    gxl1, gxr1, gmb1 = _eg1(xl, xr, mb, src1, dst1)
    pg0, p0 = _pw(gxl0, gxr0, gmb0.reshape(E0, 1), att2)
    pg1, p1 = _pw(gxl1, gxr1, gmb1.reshape(E1, 1), att2)
    num0, den0 = _es0(pg0, p0.reshape(E0), dst0)
    num1, den1 = _es1(pg1, p1.reshape(E1), dst1)

    den4 = jnp.concatenate([den0.reshape(NC, N), den1.reshape(NC, N)], 0).T
    return _final(num0, num1, den4, bias2.reshape(1, D))


# speedup vs baseline: 6.6577x; 1.0241x over previous
import functools
import jax
import jax.numpy as jnp
from jax import lax
from jax.experimental import pallas as pl
from jax.experimental.pallas import tpu as pltpu
from jax.experimental.pallas import tpu_sc as plsc

N, E, D = 10000, 320000, 128
R, NB = 8, 30
NC, NS, L = 2, 16, 16
NW = NC * NS
EPW = E // NW
CH = 80
SUP = 2000
NSUP = EPW // SUP
SUBS = SUP // CH
SP0 = 624
BLK = 400
NBLK = N // BLK

_mesh = plsc.VectorSubcoreMesh(
    core_axis_name="c", subcore_axis_name="s", num_cores=NC, num_subcores=NS)


def _proj_body(comp_ref, bases_ref, root_ref, x_ref, out_ref):
    r = pl.program_id(0)
    w = jnp.zeros((D, D), jnp.float32)
    for b in range(NB):
        w = w + comp_ref[r, b] * bases_ref[b]
    isroot = jnp.where(r == R, 1.0, 0.0)
    w = w + isroot * root_ref[...]
    out_ref[0] = jnp.dot(x_ref[...], w, preferred_element_type=jnp.float32)


def _proj(comp_pad, bases, root, x):
    return pl.pallas_call(
        _proj_body,
        grid=(R + 1, NBLK),
        in_specs=[
            pl.BlockSpec(memory_space=pltpu.SMEM),
            pl.BlockSpec((NB, D, D), lambda r, i: (0, 0, 0)),
            pl.BlockSpec((D, D), lambda r, i: (0, 0)),
            pl.BlockSpec((BLK, D), lambda r, i: (i, 0)),
        ],
        out_specs=pl.BlockSpec((1, BLK, D), lambda r, i: (r, i, 0)),
        out_shape=jax.ShapeDtypeStruct((R + 1, N, D), jnp.float32),
    )(comp_pad, bases, root, x)


def _zero_rows(rows):
    z = jnp.zeros((L,), jnp.float32)

    def body(j, _):
        def inner(k, __):
            rows[j, pl.ds(k * L, L)] = z
            return 0
        return lax.fori_loop(0, D // L, inner, 0)
    lax.fori_loop(0, CH, body, 0)


def _zero_spmem(rows, sh, s):
    for k in range(7):
        pltpu.sync_copy(rows, sh.at[pl.ds(s * SP0 + k * CH, CH)])
    pltpu.sync_copy(rows.at[pl.ds(0, 64)], sh.at[pl.ds(s * SP0 + 7 * CH, 64)])

    @pl.when(s == NS - 1)
    def _():
        pltpu.sync_copy(rows.at[pl.ds(0, 16)], sh.at[pl.ds(N - 16, 16)])


def _zero_1d(buf, n):
    z = jnp.zeros((L,), jnp.float32)

    def body(i, _):
        buf[pl.ds(i * L, L)] = z
        return 0
    lax.fori_loop(0, n // L, body, 0)


@functools.partial(
    pl.kernel,
    out_type=(jax.ShapeDtypeStruct((NC, N, D), jnp.float32),
              jax.ShapeDtypeStruct((NW, 1, N), jnp.float32)),
    mesh=_mesh,
    compiler_params=pltpu.CompilerParams(needs_layout_passes=False),
    scratch_types=[
        pltpu.VMEM((SUP,), jnp.int32),
        pltpu.VMEM((SUP,), jnp.int32),
        pltpu.VMEM((SUP,), jnp.int32),
        pltpu.VMEM((CH,), jnp.int32),
        pltpu.VMEM((CH,), jnp.int32),
        pltpu.VMEM((CH, D), jnp.float32),
        pltpu.VMEM((N,), jnp.float32),
        pltpu.VMEM_SHARED((N, D), jnp.float32),
        pltpu.SemaphoreType.DMA,
    ],
)
def _rgcn(table, src, dst, typ, agg_out, cnt_out,
          src_sup, dst_sup, typ_sup, flatb, dstb, rows, cnt_local,
          agg_sh, gsem):
    c = lax.axis_index("c")
    s = lax.axis_index("s")
    wid = s * NC + c

    _zero_rows(rows)
    _zero_spmem(rows, agg_sh, s)
    _zero_1d(cnt_local, N)
    plsc.subcore_barrier()

    base = wid * EPW
    ones = jnp.ones((L,), jnp.float32)

    def sup_body(sc, _):
        sbase = base + sc * SUP
        pltpu.sync_copy(src.at[pl.ds(sbase, SUP)], src_sup)
        pltpu.sync_copy(dst.at[pl.ds(sbase, SUP)], dst_sup)
        pltpu.sync_copy(typ.at[pl.ds(sbase, SUP)], typ_sup)

        def chunk(cc, __):
            off = cc * CH
            for g in range(CH // L):
                sl = pl.ds(off + g * L, L)
                s16 = src_sup[sl]
                t16 = typ_sup[sl]
                d16 = dst_sup[sl]
                flatb[pl.ds(g * L, L)] = t16 * N + s16
                dstb[pl.ds(g * L, L)] = d16
                plsc.addupdate_scatter(cnt_local, [d16], ones)
            pltpu.async_copy(table.at[flatb], rows, gsem).wait()
            pltpu.sync_copy(rows, agg_sh.at[dstb], add=True)
            return 0

        return lax.fori_loop(0, SUBS, chunk, 0)

    lax.fori_loop(0, NSUP, sup_body, 0)
    plsc.subcore_barrier()

    pltpu.sync_copy(agg_sh.at[pl.ds(s * SP0, SP0)],
                    agg_out.at[c, pl.ds(s * SP0, SP0)])

    @pl.when(s == NS - 1)
    def _():
        pltpu.sync_copy(agg_sh.at[pl.ds(N - 16, 16)],
                        agg_out.at[c, pl.ds(N - 16, 16)])

    pltpu.sync_copy(cnt_local, cnt_out.at[wid, 0])


def _post_body(agg_ref, cnt_ref, xroot_ref, b1_ref, ll_ref, lr_ref,
               xl_ref, xr_ref, mm_ref):
    aggsum = agg_ref[0] + agg_ref[1]
    cnt = jnp.sum(cnt_ref[...], axis=1)
    x1 = (aggsum / jnp.clip(cnt, 1.0)[:, None]
          + xroot_ref[...] + b1_ref[...])
    xl = jnp.dot(x1, ll_ref[...], preferred_element_type=jnp.float32)
    xr = jnp.dot(x1, lr_ref[...], preferred_element_type=jnp.float32)
    xl_ref[...] = xl
    xr_ref[...] = xr
    mm_ref[0, 0] = jnp.max(xl, axis=0)
    mm_ref[0, 1] = jnp.min(xl, axis=0)


def _post(agg2, cnt32, xroot, bias1, lin_l, lin_r):
    return pl.pallas_call(
        _post_body,
        grid=(NBLK,),
        in_specs=[
            pl.BlockSpec((NC, BLK, D), lambda i: (0, i, 0)),
            pl.BlockSpec((BLK, NW), lambda i: (i, 0)),
            pl.BlockSpec((BLK, D), lambda i: (i, 0)),
            pl.BlockSpec((1, D), lambda i: (0, 0)),
            pl.BlockSpec((D, D), lambda i: (0, 0)),
            pl.BlockSpec((D, D), lambda i: (0, 0)),
        ],
        out_specs=[
            pl.BlockSpec((BLK, D), lambda i: (i, 0)),
            pl.BlockSpec((BLK, D), lambda i: (i, 0)),
            pl.BlockSpec((1, 2, D), lambda i: (i, 0, 0)),
        ],
        out_shape=[
            jax.ShapeDtypeStruct((N, D), jnp.float32),
            jax.ShapeDtypeStruct((N, D), jnp.float32),
            jax.ShapeDtypeStruct((NBLK, 2, D), jnp.float32),
        ],
    )(agg2, cnt32, xroot, bias1, lin_l, lin_r)


def _bound_body(mm_ref, xr_ref, att_ref, mb_ref):
    xlmax = jnp.max(mm_ref[:, 0, :], axis=0)
    xlmin = jnp.min(mm_ref[:, 1, :], axis=0)
    attv = att_ref[0]
    ap = jnp.maximum(attv, 0.0)
    an = jnp.minimum(attv, 0.0)
    zp = xlmax[None, :] + xr_ref[...]
    zn = xlmin[None, :] + xr_ref[...]
    lrp = jnp.maximum(zp, 0.2 * zp)
    lrn = jnp.maximum(zn, 0.2 * zn)
    mb_ref[...] = jnp.sum(ap * lrp + an * lrn, axis=1, keepdims=True)


def _bound(mm, xr, att):
    return pl.pallas_call(
        _bound_body,
        grid=(NBLK,),
        in_specs=[
            pl.BlockSpec((NBLK, 2, D), lambda i: (0, 0, 0)),
            pl.BlockSpec((BLK, D), lambda i: (i, 0)),
            pl.BlockSpec((1, D), lambda i: (0, 0)),
        ],
        out_specs=pl.BlockSpec((BLK, 1), lambda i: (i, 0)),
        out_shape=jax.ShapeDtypeStruct((N, 1), jnp.float32),
    )(mm, xr, att)


def _make_egather(eh):
    epw = eh // NW
    nsup = epw // SUP

    @functools.partial(
        pl.kernel,
        out_type=(jax.ShapeDtypeStruct((eh, D), jnp.float32),
                  jax.ShapeDtypeStruct((eh, D), jnp.float32),
                  jax.ShapeDtypeStruct((eh,), jnp.float32)),
        mesh=_mesh,
        compiler_params=pltpu.CompilerParams(needs_layout_passes=False),
        scratch_types=[
            pltpu.VMEM((SUP,), jnp.int32),
            pltpu.VMEM((SUP,), jnp.int32),
            pltpu.VMEM((CH,), jnp.int32),
            pltpu.VMEM((CH,), jnp.int32),
            pltpu.VMEM((CH, D), jnp.float32),
            pltpu.VMEM((CH, D), jnp.float32),
            pltpu.VMEM((CH,), jnp.float32),
            pltpu.VMEM((N,), jnp.float32),
            pltpu.SemaphoreType.DMA,
            pltpu.SemaphoreType.DMA,
        ],
    )
    def _eg(xl, xr, mb, src, dst, gxl_out, gxr_out, gmb_out,
            src_sup, dst_sup, srcb, dstb, rows_l, rows_r, gmbb,
            mb_local, lsem, rsem):
        c = lax.axis_index("c")
        s = lax.axis_index("s")
        wid = s * NC + c

        pltpu.sync_copy(mb, mb_local)

        base = wid * epw

        def sup_body(sc, _):
            sbase = base + sc * SUP
            pltpu.sync_copy(src.at[pl.ds(sbase, SUP)], src_sup)
            pltpu.sync_copy(dst.at[pl.ds(sbase, SUP)], dst_sup)

            def chunk(cc, __):
                off = cc * CH
                for g in range(CH // L):
                    sl = pl.ds(off + g * L, L)
                    srcb[pl.ds(g * L, L)] = src_sup[sl]
                    dstb[pl.ds(g * L, L)] = dst_sup[sl]
                cl = pltpu.async_copy(xl.at[srcb], rows_l, lsem)
                cr = pltpu.async_copy(xr.at[dstb], rows_r, rsem)
                for g in range(CH // L):
                    d16 = dstb[pl.ds(g * L, L)]
                    gmbb[pl.ds(g * L, L)] = plsc.load_gather(mb_local, [d16])
                cl.wait()
                cr.wait()
                ebase = sbase + off
                pltpu.sync_copy(rows_l, gxl_out.at[pl.ds(ebase, CH)])
                pltpu.sync_copy(rows_r, gxr_out.at[pl.ds(ebase, CH)])
                pltpu.sync_copy(gmbb, gmb_out.at[pl.ds(ebase, CH)])
                return 0

            return lax.fori_loop(0, SUBS, chunk, 0)

        lax.fori_loop(0, nsup, sup_body, 0)

    return _eg


BE = 6400


def _pw_body(gxl_ref, gxr_ref, gmb_ref, att_ref, pg_ref, p_ref):
    gxl = gxl_ref[...]
    t = gxl + gxr_ref[...]
    lr = jnp.maximum(t, 0.2 * t)
    score = jnp.dot(lr, att_ref[...], preferred_element_type=jnp.float32)
    p = jnp.exp(score - gmb_ref[...])
    pg_ref[...] = p * gxl
    p_ref[...] = p


def _pw(gxl, gxr, gmb2, att2):
    eh = gxl.shape[0]
    return pl.pallas_call(
        _pw_body,
        grid=(eh // BE,),
        in_specs=[
            pl.BlockSpec((BE, D), lambda i: (i, 0)),
            pl.BlockSpec((BE, D), lambda i: (i, 0)),
            pl.BlockSpec((BE, 1), lambda i: (i, 0)),
            pl.BlockSpec((D, 1), lambda i: (0, 0)),
        ],
        out_specs=[
            pl.BlockSpec((BE, D), lambda i: (i, 0)),
            pl.BlockSpec((BE, 1), lambda i: (i, 0)),
        ],
        out_shape=[
            jax.ShapeDtypeStruct((eh, D), jnp.float32),
            jax.ShapeDtypeStruct((eh, 1), jnp.float32),
        ],
    )(gxl, gxr, gmb2, att2)


def _make_escatter(eh):
    epw = eh // NW
    nsup = epw // SUP

    @functools.partial(
        pl.kernel,
        out_type=(jax.ShapeDtypeStruct((NC, N, D), jnp.float32),
                  jax.ShapeDtypeStruct((NC, 1, N), jnp.float32)),
        mesh=_mesh,
        compiler_params=pltpu.CompilerParams(needs_layout_passes=False),
        scratch_types=[
            pltpu.VMEM((SUP,), jnp.int32),
            pltpu.VMEM((CH,), jnp.int32),
            pltpu.VMEM((CH, D), jnp.float32),
            pltpu.VMEM((CH,), jnp.float32),
            pltpu.VMEM((N,), jnp.float32),
            pltpu.VMEM_SHARED((N, D), jnp.float32),
            pltpu.VMEM_SHARED((N,), jnp.float32),
            pltpu.SemaphoreType.DMA,
        ],
    )
    def _es(pg, pvec, dst, num_out, den_out,
            dst_sup, dstb, rows, pb, zbuf, num_sh, den_sh, gsem):
        c = lax.axis_index("c")
        s = lax.axis_index("s")
        wid = s * NC + c

        _zero_rows(rows)
        _zero_spmem(rows, num_sh, s)
        _zero_1d(zbuf, N)

        @pl.when(s == 0)
        def _():
            pltpu.sync_copy(zbuf, den_sh)

        plsc.subcore_barrier()

        base = wid * epw

        def sup_body(sc, _):
            sbase = base + sc * SUP
            pltpu.sync_copy(dst.at[pl.ds(sbase, SUP)], dst_sup)

            def chunk(cc, __):
                off = cc * CH
                for g in range(CH // L):
                    dstb[pl.ds(g * L, L)] = dst_sup[pl.ds(off + g * L, L)]
                ebase = sbase + off
                cg = pltpu.async_copy(pg.at[pl.ds(ebase, CH)], rows, gsem)
                pltpu.sync_copy(pvec.at[pl.ds(ebase, CH)], pb)
                cg.wait()
                pltpu.sync_copy(pb, den_sh.at[dstb], add=True)
                pltpu.sync_copy(rows, num_sh.at[dstb], add=True)
                return 0

            return lax.fori_loop(0, SUBS, chunk, 0)

        lax.fori_loop(0, nsup, sup_body, 0)
        plsc.subcore_barrier()

        pltpu.sync_copy(num_sh.at[pl.ds(s * SP0, SP0)],
                        num_out.at[c, pl.ds(s * SP0, SP0)])

        @pl.when(s == NS - 1)
        def _():
            pltpu.sync_copy(num_sh.at[pl.ds(N - 16, 16)],
                            num_out.at[c, pl.ds(N - 16, 16)])

        @pl.when(s == 0)
        def _():
            pltpu.sync_copy(den_sh, den_out.at[c, 0])

    return _es


def _final_body(num0_ref, num1_ref, num2_ref, den_ref, b2_ref, out_ref):
    nsum = (num0_ref[0] + num0_ref[1] + num1_ref[0] + num1_ref[1]
            + num2_ref[0] + num2_ref[1])
    d = jnp.sum(den_ref[...], axis=1)[:, None]
    safe = jnp.where(d > 0.0, d, 1.0)
    out_ref[...] = jnp.where(d > 0.0, nsum / safe, 0.0) + b2_ref[...]


def _final(num0, num1, num2, den6, bias2):
    return pl.pallas_call(
        _final_body,
        grid=(NBLK,),
        in_specs=[
            pl.BlockSpec((NC, BLK, D), lambda i: (0, i, 0)),
            pl.BlockSpec((NC, BLK, D), lambda i: (0, i, 0)),
            pl.BlockSpec((NC, BLK, D), lambda i: (0, i, 0)),
            pl.BlockSpec((BLK, 3 * NC), lambda i: (i, 0)),
            pl.BlockSpec((1, D), lambda i: (0, 0)),
        ],
        out_specs=pl.BlockSpec((BLK, D), lambda i: (i, 0)),
        out_shape=jax.ShapeDtypeStruct((N, D), jnp.float32),
    )(num0, num1, num2, den6, bias2)


E0 = 128000
E1 = 128000
E2 = E - E0 - E1
_eg0 = _make_egather(E0)
_eg1 = _make_egather(E1)
_eg2 = _make_egather(E2)
_es0 = _make_escatter(E0)
_es1 = _make_escatter(E1)
_es2 = _make_escatter(E2)


def kernel(node_features, edge_index, edge_norm, edge_type, comp, bases,
           root, bias1, lin_l, lin_r, att, bias2):
    del edge_norm
    src = edge_index[0]
    dst = edge_index[1]

    comp_pad = jnp.concatenate([comp, jnp.zeros((1, NB), comp.dtype)], axis=0)
    xw_all = _proj(comp_pad, bases, root, node_features)
    table = xw_all[:R].reshape(R * N, D)
    xroot = xw_all[R]

    agg2, cnt32 = _rgcn(table, src, dst, edge_type)

    xl, xr, mm = _post(agg2, cnt32.reshape(NW, N).T, xroot,
                       bias1.reshape(1, D), lin_l, lin_r)
    mb = _bound(mm, xr, att.reshape(1, D))[:, 0]

    src0, dst0 = src[:E0], dst[:E0]
    src1, dst1 = src[E0:E0 + E1], dst[E0:E0 + E1]
    src2, dst2 = src[E0 + E1:], dst[E0 + E1:]
    att2 = att.reshape(D, 1)

    gxl0, gxr0, gmb0 = _eg0(xl, xr, mb, src0, dst0)
    gxl1, gxr1, gmb1 = _eg1(xl, xr, mb, src1, dst1)
    gxl2, gxr2, gmb2v = _eg2(xl, xr, mb, src2, dst2)
    pg0, p0 = _pw(gxl0, gxr0, gmb0.reshape(E0, 1), att2)
    pg1, p1 = _pw(gxl1, gxr1, gmb1.reshape(E1, 1), att2)
    pg2, p2 = _pw(gxl2, gxr2, gmb2v.reshape(E2, 1), att2)
    num0, den0 = _es0(pg0, p0.reshape(E0), dst0)
    num1, den1 = _es1(pg1, p1.reshape(E1), dst1)
    num2, den2 = _es2(pg2, p2.reshape(E2), dst2)

    den6 = jnp.concatenate([den0.reshape(NC, N), den1.reshape(NC, N),
                            den2.reshape(NC, N)], 0).T
    return _final(num0, num1, num2, den6, bias2.reshape(1, D))


# speedup vs baseline: 6.7543x; 1.0145x over previous
import functools
import jax
import jax.numpy as jnp
from jax import lax
from jax.experimental import pallas as pl
from jax.experimental.pallas import tpu as pltpu
from jax.experimental.pallas import tpu_sc as plsc

N, E, D = 10000, 320000, 128
R, NB = 8, 30
NC, NS, L = 2, 16, 16
NW = NC * NS
EPW = E // NW
CH = 80
SUP = 2000
NSUP = EPW // SUP
SUBS = SUP // CH
SP0 = 624
BLK = 400
NBLK = N // BLK

_mesh = plsc.VectorSubcoreMesh(
    core_axis_name="c", subcore_axis_name="s", num_cores=NC, num_subcores=NS)


def _proj_body(comp_ref, bases_ref, root_ref, x_ref, out_ref):
    r = pl.program_id(0)
    w = jnp.zeros((D, D), jnp.float32)
    for b in range(NB):
        w = w + comp_ref[r, b] * bases_ref[b]
    isroot = jnp.where(r == R, 1.0, 0.0)
    w = w + isroot * root_ref[...]
    out_ref[0] = jnp.dot(x_ref[...], w, preferred_element_type=jnp.float32)


def _proj(comp_pad, bases, root, x):
    return pl.pallas_call(
        _proj_body,
        grid=(R + 1, NBLK),
        in_specs=[
            pl.BlockSpec(memory_space=pltpu.SMEM),
            pl.BlockSpec((NB, D, D), lambda r, i: (0, 0, 0)),
            pl.BlockSpec((D, D), lambda r, i: (0, 0)),
            pl.BlockSpec((BLK, D), lambda r, i: (i, 0)),
        ],
        out_specs=pl.BlockSpec((1, BLK, D), lambda r, i: (r, i, 0)),
        out_shape=jax.ShapeDtypeStruct((R + 1, N, D), jnp.float32),
    )(comp_pad, bases, root, x)


def _zero_rows(rows):
    z = jnp.zeros((L,), jnp.float32)

    def body(j, _):
        def inner(k, __):
            rows[j, pl.ds(k * L, L)] = z
            return 0
        return lax.fori_loop(0, D // L, inner, 0)
    lax.fori_loop(0, CH, body, 0)


def _zero_spmem(rows, sh, s):
    for k in range(7):
        pltpu.sync_copy(rows, sh.at[pl.ds(s * SP0 + k * CH, CH)])
    pltpu.sync_copy(rows.at[pl.ds(0, 64)], sh.at[pl.ds(s * SP0 + 7 * CH, 64)])

    @pl.when(s == NS - 1)
    def _():
        pltpu.sync_copy(rows.at[pl.ds(0, 16)], sh.at[pl.ds(N - 16, 16)])


def _zero_1d(buf, n):
    z = jnp.zeros((L,), jnp.float32)

    def body(i, _):
        buf[pl.ds(i * L, L)] = z
        return 0
    lax.fori_loop(0, n // L, body, 0)


@functools.partial(
    pl.kernel,
    out_type=(jax.ShapeDtypeStruct((NC, N, D), jnp.float32),
              jax.ShapeDtypeStruct((NW, 1, N), jnp.float32)),
    mesh=_mesh,
    compiler_params=pltpu.CompilerParams(needs_layout_passes=False),
    scratch_types=[
        pltpu.VMEM((SUP,), jnp.int32),
        pltpu.VMEM((SUP,), jnp.int32),
        pltpu.VMEM((SUP,), jnp.int32),
        pltpu.VMEM((CH,), jnp.int32),
        pltpu.VMEM((CH,), jnp.int32),
        pltpu.VMEM((CH, D), jnp.float32),
        pltpu.VMEM((N,), jnp.float32),
        pltpu.VMEM_SHARED((N, D), jnp.float32),
        pltpu.SemaphoreType.DMA,
    ],
)
def _rgcn(table, src, dst, typ, agg_out, cnt_out,
          src_sup, dst_sup, typ_sup, flatb, dstb, rows, cnt_local,
          agg_sh, gsem):
    c = lax.axis_index("c")
    s = lax.axis_index("s")
    wid = s * NC + c

    _zero_rows(rows)
    _zero_spmem(rows, agg_sh, s)
    _zero_1d(cnt_local, N)
    plsc.subcore_barrier()

    base = wid * EPW
    ones = jnp.ones((L,), jnp.float32)

    def sup_body(sc, _):
        sbase = base + sc * SUP
        pltpu.sync_copy(src.at[pl.ds(sbase, SUP)], src_sup)
        pltpu.sync_copy(dst.at[pl.ds(sbase, SUP)], dst_sup)
        pltpu.sync_copy(typ.at[pl.ds(sbase, SUP)], typ_sup)

        def chunk(cc, __):
            off = cc * CH
            for g in range(CH // L):
                sl = pl.ds(off + g * L, L)
                s16 = src_sup[sl]
                t16 = typ_sup[sl]
                d16 = dst_sup[sl]
                flatb[pl.ds(g * L, L)] = t16 * N + s16
                dstb[pl.ds(g * L, L)] = d16
                plsc.addupdate_scatter(cnt_local, [d16], ones)
            pltpu.async_copy(table.at[flatb], rows, gsem).wait()
            pltpu.sync_copy(rows, agg_sh.at[dstb], add=True)
            return 0

        return lax.fori_loop(0, SUBS, chunk, 0)

    lax.fori_loop(0, NSUP, sup_body, 0)
    plsc.subcore_barrier()

    pltpu.sync_copy(agg_sh.at[pl.ds(s * SP0, SP0)],
                    agg_out.at[c, pl.ds(s * SP0, SP0)])

    @pl.when(s == NS - 1)
    def _():
        pltpu.sync_copy(agg_sh.at[pl.ds(N - 16, 16)],
                        agg_out.at[c, pl.ds(N - 16, 16)])

    pltpu.sync_copy(cnt_local, cnt_out.at[wid, 0])


def _post_body(agg_ref, cnt_ref, xroot_ref, b1_ref, ll_ref, lr_ref,
               xl_ref, xr_ref, mm_ref):
    aggsum = agg_ref[0] + agg_ref[1]
    cnt = jnp.sum(cnt_ref[...], axis=1)
    x1 = (aggsum / jnp.clip(cnt, 1.0)[:, None]
          + xroot_ref[...] + b1_ref[...])
    xl = jnp.dot(x1, ll_ref[...], preferred_element_type=jnp.float32)
    xr = jnp.dot(x1, lr_ref[...], preferred_element_type=jnp.float32)
    xl_ref[...] = xl
    xr_ref[...] = xr
    mm_ref[0, 0] = jnp.max(xl, axis=0)
    mm_ref[0, 1] = jnp.min(xl, axis=0)


def _post(agg2, cnt32, xroot, bias1, lin_l, lin_r):
    return pl.pallas_call(
        _post_body,
        grid=(NBLK,),
        in_specs=[
            pl.BlockSpec((NC, BLK, D), lambda i: (0, i, 0)),
            pl.BlockSpec((BLK, NW), lambda i: (i, 0)),
            pl.BlockSpec((BLK, D), lambda i: (i, 0)),
            pl.BlockSpec((1, D), lambda i: (0, 0)),
            pl.BlockSpec((D, D), lambda i: (0, 0)),
            pl.BlockSpec((D, D), lambda i: (0, 0)),
        ],
        out_specs=[
            pl.BlockSpec((BLK, D), lambda i: (i, 0)),
            pl.BlockSpec((BLK, D), lambda i: (i, 0)),
            pl.BlockSpec((1, 2, D), lambda i: (i, 0, 0)),
        ],
        out_shape=[
            jax.ShapeDtypeStruct((N, D), jnp.float32),
            jax.ShapeDtypeStruct((N, D), jnp.float32),
            jax.ShapeDtypeStruct((NBLK, 2, D), jnp.float32),
        ],
    )(agg2, cnt32, xroot, bias1, lin_l, lin_r)


def _bound_body(mm_ref, xr_ref, att_ref, mb_ref):
    xlmax = jnp.max(mm_ref[:, 0, :], axis=0)
    xlmin = jnp.min(mm_ref[:, 1, :], axis=0)
    attv = att_ref[0]
    ap = jnp.maximum(attv, 0.0)
    an = jnp.minimum(attv, 0.0)
    zp = xlmax[None, :] + xr_ref[...]
    zn = xlmin[None, :] + xr_ref[...]
    lrp = jnp.maximum(zp, 0.2 * zp)
    lrn = jnp.maximum(zn, 0.2 * zn)
    mb_ref[...] = jnp.sum(ap * lrp + an * lrn, axis=1, keepdims=True)


def _bound(mm, xr, att):
    return pl.pallas_call(
        _bound_body,
        grid=(NBLK,),
        in_specs=[
            pl.BlockSpec((NBLK, 2, D), lambda i: (0, 0, 0)),
            pl.BlockSpec((BLK, D), lambda i: (i, 0)),
            pl.BlockSpec((1, D), lambda i: (0, 0)),
        ],
        out_specs=pl.BlockSpec((BLK, 1), lambda i: (i, 0)),
        out_shape=jax.ShapeDtypeStruct((N, 1), jnp.float32),
    )(mm, xr, att)


def _make_egather(eh):
    epw = eh // NW
    nsup = epw // SUP

    @functools.partial(
        pl.kernel,
        out_type=(jax.ShapeDtypeStruct((eh, D), jnp.float32),
                  jax.ShapeDtypeStruct((eh, D), jnp.float32),
                  jax.ShapeDtypeStruct((eh,), jnp.float32)),
        mesh=_mesh,
        compiler_params=pltpu.CompilerParams(needs_layout_passes=False),
        scratch_types=[
            pltpu.VMEM((SUP,), jnp.int32),
            pltpu.VMEM((SUP,), jnp.int32),
            pltpu.VMEM((CH,), jnp.int32),
            pltpu.VMEM((CH,), jnp.int32),
            pltpu.VMEM((2, CH, D), jnp.float32),
            pltpu.VMEM((2, CH, D), jnp.float32),
            pltpu.VMEM((2, CH), jnp.float32),
            pltpu.VMEM((N,), jnp.float32),
            pltpu.SemaphoreType.DMA,
            pltpu.SemaphoreType.DMA,
            pltpu.SemaphoreType.DMA,
            pltpu.SemaphoreType.DMA,
        ],
    )
    def _eg(xl, xr, mb, src, dst, gxl_out, gxr_out, gmb_out,
            src_sup, dst_sup, srcb, dstb, rows_l2, rows_r2, gmbb2,
            mb_local, lsem, rsem, wsem_a, wsem_b):
        c = lax.axis_index("c")
        s = lax.axis_index("s")
        wid = s * NC + c

        pltpu.sync_copy(mb, mb_local)

        base = wid * epw

        def sup_body(sup_i, _):
            sbase = base + sup_i * SUP
            pltpu.sync_copy(src.at[pl.ds(sbase, SUP)], src_sup)
            pltpu.sync_copy(dst.at[pl.ds(sbase, SUP)], dst_sup)

            def chunk(cc, __):
                t = sup_i * SUBS + cc
                off = cc * CH
                for g in range(CH // L):
                    sl = pl.ds(off + g * L, L)
                    srcb[pl.ds(g * L, L)] = src_sup[sl]
                    dstb[pl.ds(g * L, L)] = dst_sup[sl]
                ebase = sbase + off
                dl = gxl_out.at[pl.ds(ebase, CH)]
                dr = gxr_out.at[pl.ds(ebase, CH)]
                dm = gmb_out.at[pl.ds(ebase, CH)]

                def go(rows_l, rows_r, gmbb, wsem):
                    @pl.when(t >= 2)
                    def _():
                        pltpu.make_async_copy(rows_l, dl, wsem).wait()
                        pltpu.make_async_copy(rows_r, dr, wsem).wait()
                        pltpu.make_async_copy(gmbb, dm, wsem).wait()
                    cl = pltpu.async_copy(xl.at[srcb], rows_l, lsem)
                    cr = pltpu.async_copy(xr.at[dstb], rows_r, rsem)
                    for g in range(CH // L):
                        d16 = dstb[pl.ds(g * L, L)]
                        gmbb[pl.ds(g * L, L)] = plsc.load_gather(
                            mb_local, [d16])
                    cl.wait()
                    cr.wait()
                    pltpu.async_copy(rows_l, dl, wsem)
                    pltpu.async_copy(rows_r, dr, wsem)
                    pltpu.async_copy(gmbb, dm, wsem)

                @pl.when(t % 2 == 0)
                def _():
                    go(rows_l2.at[0], rows_r2.at[0], gmbb2.at[0], wsem_a)

                @pl.when(t % 2 == 1)
                def _():
                    go(rows_l2.at[1], rows_r2.at[1], gmbb2.at[1], wsem_b)

                return 0

            return lax.fori_loop(0, SUBS, chunk, 0)

        lax.fori_loop(0, nsup, sup_body, 0)

        dl = gxl_out.at[pl.ds(base, CH)]
        dr = gxr_out.at[pl.ds(base, CH)]
        dm = gmb_out.at[pl.ds(base, CH)]
        for b, wsem in ((0, wsem_a), (1, wsem_b)):
            pltpu.make_async_copy(rows_l2.at[b], dl, wsem).wait()
            pltpu.make_async_copy(rows_r2.at[b], dr, wsem).wait()
            pltpu.make_async_copy(gmbb2.at[b], dm, wsem).wait()

    return _eg


BE = 6400


def _pw_body(gxl_ref, gxr_ref, gmb_ref, att_ref, pg_ref, p_ref):
    gxl = gxl_ref[...]
    t = gxl + gxr_ref[...]
    lr = jnp.maximum(t, 0.2 * t)
    score = jnp.dot(lr, att_ref[...], preferred_element_type=jnp.float32)
    p = jnp.exp(score - gmb_ref[...])
    pg_ref[...] = p * gxl
    p_ref[...] = p


def _pw(gxl, gxr, gmb2, att2):
    eh = gxl.shape[0]
    return pl.pallas_call(
        _pw_body,
        grid=(eh // BE,),
        in_specs=[
            pl.BlockSpec((BE, D), lambda i: (i, 0)),
            pl.BlockSpec((BE, D), lambda i: (i, 0)),
            pl.BlockSpec((BE, 1), lambda i: (i, 0)),
            pl.BlockSpec((D, 1), lambda i: (0, 0)),
        ],
        out_specs=[
            pl.BlockSpec((BE, D), lambda i: (i, 0)),
            pl.BlockSpec((BE, 1), lambda i: (i, 0)),
        ],
        out_shape=[
            jax.ShapeDtypeStruct((eh, D), jnp.float32),
            jax.ShapeDtypeStruct((eh, 1), jnp.float32),
        ],
    )(gxl, gxr, gmb2, att2)


def _make_escatter(eh):
    epw = eh // NW
    nsup = epw // SUP

    @functools.partial(
        pl.kernel,
        out_type=(jax.ShapeDtypeStruct((NC, N, D), jnp.float32),
                  jax.ShapeDtypeStruct((NC, 1, N), jnp.float32)),
        mesh=_mesh,
        compiler_params=pltpu.CompilerParams(needs_layout_passes=False),
        scratch_types=[
            pltpu.VMEM((SUP,), jnp.int32),
            pltpu.VMEM((CH,), jnp.int32),
            pltpu.VMEM((CH, D), jnp.float32),
            pltpu.VMEM((CH,), jnp.float32),
            pltpu.VMEM((N,), jnp.float32),
            pltpu.VMEM_SHARED((N, D), jnp.float32),
            pltpu.VMEM_SHARED((N,), jnp.float32),
            pltpu.SemaphoreType.DMA,
        ],
    )
    def _es(pg, pvec, dst, num_out, den_out,
            dst_sup, dstb, rows, pb, zbuf, num_sh, den_sh, gsem):
        c = lax.axis_index("c")
        s = lax.axis_index("s")
        wid = s * NC + c

        _zero_rows(rows)
        _zero_spmem(rows, num_sh, s)
        _zero_1d(zbuf, N)

        @pl.when(s == 0)
        def _():
            pltpu.sync_copy(zbuf, den_sh)

        plsc.subcore_barrier()

        base = wid * epw

        def sup_body(sc, _):
            sbase = base + sc * SUP
            pltpu.sync_copy(dst.at[pl.ds(sbase, SUP)], dst_sup)

            def chunk(cc, __):
                off = cc * CH
                for g in range(CH // L):
                    dstb[pl.ds(g * L, L)] = dst_sup[pl.ds(off + g * L, L)]
                ebase = sbase + off
                cg = pltpu.async_copy(pg.at[pl.ds(ebase, CH)], rows, gsem)
                pltpu.sync_copy(pvec.at[pl.ds(ebase, CH)], pb)
                cg.wait()
                pltpu.sync_copy(pb, den_sh.at[dstb], add=True)
                pltpu.sync_copy(rows, num_sh.at[dstb], add=True)
                return 0

            return lax.fori_loop(0, SUBS, chunk, 0)

        lax.fori_loop(0, nsup, sup_body, 0)
        plsc.subcore_barrier()

        pltpu.sync_copy(num_sh.at[pl.ds(s * SP0, SP0)],
                        num_out.at[c, pl.ds(s * SP0, SP0)])

        @pl.when(s == NS - 1)
        def _():
            pltpu.sync_copy(num_sh.at[pl.ds(N - 16, 16)],
                            num_out.at[c, pl.ds(N - 16, 16)])

        @pl.when(s == 0)
        def _():
            pltpu.sync_copy(den_sh, den_out.at[c, 0])

    return _es


def _final_body(num0_ref, num1_ref, num2_ref, den_ref, b2_ref, out_ref):
    nsum = (num0_ref[0] + num0_ref[1] + num1_ref[0] + num1_ref[1]
            + num2_ref[0] + num2_ref[1])
    d = jnp.sum(den_ref[...], axis=1)[:, None]
    safe = jnp.where(d > 0.0, d, 1.0)
    out_ref[...] = jnp.where(d > 0.0, nsum / safe, 0.0) + b2_ref[...]


def _final(num0, num1, num2, den6, bias2):
    return pl.pallas_call(
        _final_body,
        grid=(NBLK,),
        in_specs=[
            pl.BlockSpec((NC, BLK, D), lambda i: (0, i, 0)),
            pl.BlockSpec((NC, BLK, D), lambda i: (0, i, 0)),
            pl.BlockSpec((NC, BLK, D), lambda i: (0, i, 0)),
            pl.BlockSpec((BLK, 3 * NC), lambda i: (i, 0)),
            pl.BlockSpec((1, D), lambda i: (0, 0)),
        ],
        out_specs=pl.BlockSpec((BLK, D), lambda i: (i, 0)),
        out_shape=jax.ShapeDtypeStruct((N, D), jnp.float32),
    )(num0, num1, num2, den6, bias2)


E0 = 128000
E1 = 128000
E2 = E - E0 - E1
_eg0 = _make_egather(E0)
_eg1 = _make_egather(E1)
_eg2 = _make_egather(E2)
_es0 = _make_escatter(E0)
_es1 = _make_escatter(E1)
_es2 = _make_escatter(E2)


def kernel(node_features, edge_index, edge_norm, edge_type, comp, bases,
           root, bias1, lin_l, lin_r, att, bias2):
    del edge_norm
    src = edge_index[0]
    dst = edge_index[1]

    comp_pad = jnp.concatenate([comp, jnp.zeros((1, NB), comp.dtype)], axis=0)
    xw_all = _proj(comp_pad, bases, root, node_features)
    table = xw_all[:R].reshape(R * N, D)
    xroot = xw_all[R]

    agg2, cnt32 = _rgcn(table, src, dst, edge_type)

    xl, xr, mm = _post(agg2, cnt32.reshape(NW, N).T, xroot,
                       bias1.reshape(1, D), lin_l, lin_r)
    mb = _bound(mm, xr, att.reshape(1, D))[:, 0]

    src0, dst0 = src[:E0], dst[:E0]
    src1, dst1 = src[E0:E0 + E1], dst[E0:E0 + E1]
    src2, dst2 = src[E0 + E1:], dst[E0 + E1:]
    att2 = att.reshape(D, 1)

    gxl0, gxr0, gmb0 = _eg0(xl, xr, mb, src0, dst0)
    gxl1, gxr1, gmb1 = _eg1(xl, xr, mb, src1, dst1)
    gxl2, gxr2, gmb2v = _eg2(xl, xr, mb, src2, dst2)
    pg0, p0 = _pw(gxl0, gxr0, gmb0.reshape(E0, 1), att2)
    pg1, p1 = _pw(gxl1, gxr1, gmb1.reshape(E1, 1), att2)
    pg2, p2 = _pw(gxl2, gxr2, gmb2v.reshape(E2, 1), att2)
    num0, den0 = _es0(pg0, p0.reshape(E0), dst0)
    num1, den1 = _es1(pg1, p1.reshape(E1), dst1)
    num2, den2 = _es2(pg2, p2.reshape(E2), dst2)

    den6 = jnp.concatenate([den0.reshape(NC, N), den1.reshape(NC, N),
                            den2.reshape(NC, N)], 0).T
    return _final(num0, num1, num2, den6, bias2.reshape(1, D))


# speedup vs baseline: 6.8808x; 1.0187x over previous
import functools
import jax
import jax.numpy as jnp
from jax import lax
from jax.experimental import pallas as pl
from jax.experimental.pallas import tpu as pltpu
from jax.experimental.pallas import tpu_sc as plsc

N, E, D = 10000, 320000, 128
R, NB = 8, 30
NC, NS, L = 2, 16, 16
NW = NC * NS
EPW = E // NW
CH = 80
SUP = 2000
NSUP = EPW // SUP
SUBS = SUP // CH
SP0 = 624
BLK = 400
NBLK = N // BLK

_mesh = plsc.VectorSubcoreMesh(
    core_axis_name="c", subcore_axis_name="s", num_cores=NC, num_subcores=NS)


def _proj_body(comp_ref, bases_ref, root_ref, x_ref, out_ref):
    r = pl.program_id(0)
    w = jnp.zeros((D, D), jnp.float32)
    for b in range(NB):
        w = w + comp_ref[r, b] * bases_ref[b]
    isroot = jnp.where(r == R, 1.0, 0.0)
    w = w + isroot * root_ref[...]
    out_ref[0] = jnp.dot(x_ref[...], w, preferred_element_type=jnp.float32)


def _proj(comp_pad, bases, root, x):
    return pl.pallas_call(
        _proj_body,
        grid=(R + 1, NBLK),
        in_specs=[
            pl.BlockSpec(memory_space=pltpu.SMEM),
            pl.BlockSpec((NB, D, D), lambda r, i: (0, 0, 0)),
            pl.BlockSpec((D, D), lambda r, i: (0, 0)),
            pl.BlockSpec((BLK, D), lambda r, i: (i, 0)),
        ],
        out_specs=pl.BlockSpec((1, BLK, D), lambda r, i: (r, i, 0)),
        out_shape=jax.ShapeDtypeStruct((R + 1, N, D), jnp.float32),
    )(comp_pad, bases, root, x)


def _zero_rows(rows):
    z = jnp.zeros((L,), jnp.float32)

    def body(j, _):
        def inner(k, __):
            rows[j, pl.ds(k * L, L)] = z
            return 0
        return lax.fori_loop(0, D // L, inner, 0)
    lax.fori_loop(0, CH, body, 0)


def _zero_spmem(rows, sh, s):
    for k in range(7):
        pltpu.sync_copy(rows, sh.at[pl.ds(s * SP0 + k * CH, CH)])
    pltpu.sync_copy(rows.at[pl.ds(0, 64)], sh.at[pl.ds(s * SP0 + 7 * CH, 64)])

    @pl.when(s == NS - 1)
    def _():
        pltpu.sync_copy(rows.at[pl.ds(0, 16)], sh.at[pl.ds(N - 16, 16)])


def _zero_1d(buf, n):
    z = jnp.zeros((L,), jnp.float32)

    def body(i, _):
        buf[pl.ds(i * L, L)] = z
        return 0
    lax.fori_loop(0, n // L, body, 0)


@functools.partial(
    pl.kernel,
    out_type=(jax.ShapeDtypeStruct((NC, N, D), jnp.float32),
              jax.ShapeDtypeStruct((NW, 1, N), jnp.float32)),
    mesh=_mesh,
    compiler_params=pltpu.CompilerParams(needs_layout_passes=False),
    scratch_types=[
        pltpu.VMEM((SUP,), jnp.int32),
        pltpu.VMEM((SUP,), jnp.int32),
        pltpu.VMEM((SUP,), jnp.int32),
        pltpu.VMEM((CH,), jnp.int32),
        pltpu.VMEM((CH,), jnp.int32),
        pltpu.VMEM((CH, D), jnp.float32),
        pltpu.VMEM((N,), jnp.float32),
        pltpu.VMEM_SHARED((N, D), jnp.float32),
        pltpu.SemaphoreType.DMA,
    ],
)
def _rgcn(table, src, dst, typ, agg_out, cnt_out,
          src_sup, dst_sup, typ_sup, flatb, dstb, rows, cnt_local,
          agg_sh, gsem):
    c = lax.axis_index("c")
    s = lax.axis_index("s")
    wid = s * NC + c

    _zero_rows(rows)
    _zero_spmem(rows, agg_sh, s)
    _zero_1d(cnt_local, N)
    plsc.subcore_barrier()

    base = wid * EPW
    ones = jnp.ones((L,), jnp.float32)

    def sup_body(sc, _):
        sbase = base + sc * SUP
        pltpu.sync_copy(src.at[pl.ds(sbase, SUP)], src_sup)
        pltpu.sync_copy(dst.at[pl.ds(sbase, SUP)], dst_sup)
        pltpu.sync_copy(typ.at[pl.ds(sbase, SUP)], typ_sup)

        def chunk(cc, __):
            off = cc * CH
            for g in range(CH // L):
                sl = pl.ds(off + g * L, L)
                s16 = src_sup[sl]
                t16 = typ_sup[sl]
                d16 = dst_sup[sl]
                flatb[pl.ds(g * L, L)] = t16 * N + s16
                dstb[pl.ds(g * L, L)] = d16
                plsc.addupdate_scatter(cnt_local, [d16], ones)
            pltpu.async_copy(table.at[flatb], rows, gsem).wait()
            pltpu.sync_copy(rows, agg_sh.at[dstb], add=True)
            return 0

        return lax.fori_loop(0, SUBS, chunk, 0)

    lax.fori_loop(0, NSUP, sup_body, 0)
    plsc.subcore_barrier()

    pltpu.sync_copy(agg_sh.at[pl.ds(s * SP0, SP0)],
                    agg_out.at[c, pl.ds(s * SP0, SP0)])

    @pl.when(s == NS - 1)
    def _():
        pltpu.sync_copy(agg_sh.at[pl.ds(N - 16, 16)],
                        agg_out.at[c, pl.ds(N - 16, 16)])

    pltpu.sync_copy(cnt_local, cnt_out.at[wid, 0])


def _post_body(agg_ref, cnt_ref, xroot_ref, b1_ref, ll_ref, lr_ref,
               xl_ref, xr_ref, mm_ref):
    aggsum = agg_ref[0] + agg_ref[1]
    cnt = jnp.sum(cnt_ref[...], axis=1)
    x1 = (aggsum / jnp.clip(cnt, 1.0)[:, None]
          + xroot_ref[...] + b1_ref[...])
    xl = jnp.dot(x1, ll_ref[...], preferred_element_type=jnp.float32)
    xr = jnp.dot(x1, lr_ref[...], preferred_element_type=jnp.float32)
    xl_ref[...] = xl
    xr_ref[...] = xr
    mm_ref[0, 0] = jnp.max(xl, axis=0)
    mm_ref[0, 1] = jnp.min(xl, axis=0)


def _post(agg2, cnt32, xroot, bias1, lin_l, lin_r):
    return pl.pallas_call(
        _post_body,
        grid=(NBLK,),
        in_specs=[
            pl.BlockSpec((NC, BLK, D), lambda i: (0, i, 0)),
            pl.BlockSpec((BLK, NW), lambda i: (i, 0)),
            pl.BlockSpec((BLK, D), lambda i: (i, 0)),
            pl.BlockSpec((1, D), lambda i: (0, 0)),
            pl.BlockSpec((D, D), lambda i: (0, 0)),
            pl.BlockSpec((D, D), lambda i: (0, 0)),
        ],
        out_specs=[
            pl.BlockSpec((BLK, D), lambda i: (i, 0)),
            pl.BlockSpec((BLK, D), lambda i: (i, 0)),
            pl.BlockSpec((1, 2, D), lambda i: (i, 0, 0)),
        ],
        out_shape=[
            jax.ShapeDtypeStruct((N, D), jnp.float32),
            jax.ShapeDtypeStruct((N, D), jnp.float32),
            jax.ShapeDtypeStruct((NBLK, 2, D), jnp.float32),
        ],
    )(agg2, cnt32, xroot, bias1, lin_l, lin_r)


def _bound_body(mm_ref, xr_ref, att_ref, mb_ref):
    xlmax = jnp.max(mm_ref[:, 0, :], axis=0)
    xlmin = jnp.min(mm_ref[:, 1, :], axis=0)
    attv = att_ref[0]
    ap = jnp.maximum(attv, 0.0)
    an = jnp.minimum(attv, 0.0)
    zp = xlmax[None, :] + xr_ref[...]
    zn = xlmin[None, :] + xr_ref[...]
    lrp = jnp.maximum(zp, 0.2 * zp)
    lrn = jnp.maximum(zn, 0.2 * zn)
    mb_ref[...] = jnp.sum(ap * lrp + an * lrn, axis=1, keepdims=True)


def _bound(mm, xr, att):
    return pl.pallas_call(
        _bound_body,
        grid=(NBLK,),
        in_specs=[
            pl.BlockSpec((NBLK, 2, D), lambda i: (0, 0, 0)),
            pl.BlockSpec((BLK, D), lambda i: (i, 0)),
            pl.BlockSpec((1, D), lambda i: (0, 0)),
        ],
        out_specs=pl.BlockSpec((BLK, 1), lambda i: (i, 0)),
        out_shape=jax.ShapeDtypeStruct((N, 1), jnp.float32),
    )(mm, xr, att)


def _make_egather(eh):
    epw = eh // NW
    nsup = epw // SUP

    @functools.partial(
        pl.kernel,
        out_type=(jax.ShapeDtypeStruct((eh, D), jnp.float32),
                  jax.ShapeDtypeStruct((eh, D), jnp.float32),
                  jax.ShapeDtypeStruct((eh,), jnp.float32)),
        mesh=_mesh,
        compiler_params=pltpu.CompilerParams(needs_layout_passes=False),
        scratch_types=[
            pltpu.VMEM((SUP,), jnp.int32),
            pltpu.VMEM((SUP,), jnp.int32),
            pltpu.VMEM((CH,), jnp.int32),
            pltpu.VMEM((CH,), jnp.int32),
            pltpu.VMEM((2, CH, D), jnp.float32),
            pltpu.VMEM((2, CH, D), jnp.float32),
            pltpu.VMEM((2, CH), jnp.float32),
            pltpu.VMEM((N,), jnp.float32),
            pltpu.SemaphoreType.DMA,
            pltpu.SemaphoreType.DMA,
            pltpu.SemaphoreType.DMA,
            pltpu.SemaphoreType.DMA,
        ],
    )
    def _eg(xl, xr, mb, src, dst, gxl_out, gxr_out, gmb_out,
            src_sup, dst_sup, srcb, dstb, rows_l2, rows_r2, gmbb2,
            mb_local, lsem, rsem, wsem_a, wsem_b):
        c = lax.axis_index("c")
        s = lax.axis_index("s")
        wid = s * NC + c

        pltpu.sync_copy(mb, mb_local)

        base = wid * epw

        def sup_body(sup_i, _):
            sbase = base + sup_i * SUP
            pltpu.sync_copy(src.at[pl.ds(sbase, SUP)], src_sup)
            pltpu.sync_copy(dst.at[pl.ds(sbase, SUP)], dst_sup)

            def chunk(cc, __):
                t = sup_i * SUBS + cc
                off = cc * CH
                for g in range(CH // L):
                    sl = pl.ds(off + g * L, L)
                    srcb[pl.ds(g * L, L)] = src_sup[sl]
                    dstb[pl.ds(g * L, L)] = dst_sup[sl]
                ebase = sbase + off
                dl = gxl_out.at[pl.ds(ebase, CH)]
                dr = gxr_out.at[pl.ds(ebase, CH)]
                dm = gmb_out.at[pl.ds(ebase, CH)]

                def go(rows_l, rows_r, gmbb, wsem):
                    @pl.when(t >= 2)
                    def _():
                        pltpu.make_async_copy(rows_l, dl, wsem).wait()
                        pltpu.make_async_copy(rows_r, dr, wsem).wait()
                        pltpu.make_async_copy(gmbb, dm, wsem).wait()
                    cl = pltpu.async_copy(xl.at[srcb], rows_l, lsem)
                    cr = pltpu.async_copy(xr.at[dstb], rows_r, rsem)
                    for g in range(CH // L):
                        d16 = dstb[pl.ds(g * L, L)]
                        gmbb[pl.ds(g * L, L)] = plsc.load_gather(
                            mb_local, [d16])
                    cl.wait()
                    cr.wait()
                    pltpu.async_copy(rows_l, dl, wsem)
                    pltpu.async_copy(rows_r, dr, wsem)
                    pltpu.async_copy(gmbb, dm, wsem)

                @pl.when(t % 2 == 0)
                def _():
                    go(rows_l2.at[0], rows_r2.at[0], gmbb2.at[0], wsem_a)

                @pl.when(t % 2 == 1)
                def _():
                    go(rows_l2.at[1], rows_r2.at[1], gmbb2.at[1], wsem_b)

                return 0

            return lax.fori_loop(0, SUBS, chunk, 0)

        lax.fori_loop(0, nsup, sup_body, 0)

        dl = gxl_out.at[pl.ds(base, CH)]
        dr = gxr_out.at[pl.ds(base, CH)]
        dm = gmb_out.at[pl.ds(base, CH)]
        for b, wsem in ((0, wsem_a), (1, wsem_b)):
            pltpu.make_async_copy(rows_l2.at[b], dl, wsem).wait()
            pltpu.make_async_copy(rows_r2.at[b], dr, wsem).wait()
            pltpu.make_async_copy(gmbb2.at[b], dm, wsem).wait()

    return _eg


BE = 6400


def _pw_body(gxl_ref, gxr_ref, gmb_ref, att_ref, pg_ref, p_ref):
    gxl = gxl_ref[...]
    t = gxl + gxr_ref[...]
    lr = jnp.maximum(t, 0.2 * t)
    score = jnp.dot(lr, att_ref[...], preferred_element_type=jnp.float32)
    p = jnp.exp(score - gmb_ref[...])
    pg_ref[...] = p * gxl
    p_ref[...] = p


def _pw(gxl, gxr, gmb2, att2):
    eh = gxl.shape[0]
    return pl.pallas_call(
        _pw_body,
        grid=(eh // BE,),
        in_specs=[
            pl.BlockSpec((BE, D), lambda i: (i, 0)),
            pl.BlockSpec((BE, D), lambda i: (i, 0)),
            pl.BlockSpec((BE, 1), lambda i: (i, 0)),
            pl.BlockSpec((D, 1), lambda i: (0, 0)),
        ],
        out_specs=[
            pl.BlockSpec((BE, D), lambda i: (i, 0)),
            pl.BlockSpec((BE, 1), lambda i: (i, 0)),
        ],
        out_shape=[
            jax.ShapeDtypeStruct((eh, D), jnp.float32),
            jax.ShapeDtypeStruct((eh, 1), jnp.float32),
        ],
    )(gxl, gxr, gmb2, att2)


def _make_escatter(eh):
    epw = eh // NW
    nsup = epw // SUP

    @functools.partial(
        pl.kernel,
        out_type=(jax.ShapeDtypeStruct((NC, N, D), jnp.float32),
                  jax.ShapeDtypeStruct((NC, 1, N), jnp.float32)),
        mesh=_mesh,
        compiler_params=pltpu.CompilerParams(needs_layout_passes=False),
        scratch_types=[
            pltpu.VMEM((SUP,), jnp.int32),
            pltpu.VMEM((2, CH), jnp.int32),
            pltpu.VMEM((2, CH, D), jnp.float32),
            pltpu.VMEM((2, CH), jnp.float32),
            pltpu.VMEM((N,), jnp.float32),
            pltpu.VMEM_SHARED((N, D), jnp.float32),
            pltpu.VMEM_SHARED((N,), jnp.float32),
            pltpu.SemaphoreType.DMA,
            pltpu.SemaphoreType.DMA,
            pltpu.SemaphoreType.DMA,
        ],
    )
    def _es(pg, pvec, dst, num_out, den_out,
            dst_sup, dstb2, rows2, pb2, zbuf, num_sh, den_sh,
            gsem, ssem_a, ssem_b):
        c = lax.axis_index("c")
        s = lax.axis_index("s")
        wid = s * NC + c

        _zero_rows(rows2.at[0])
        _zero_spmem(rows2.at[0], num_sh, s)
        _zero_rows(rows2.at[1])
        _zero_1d(zbuf, N)

        @pl.when(s == 0)
        def _():
            pltpu.sync_copy(zbuf, den_sh)

        plsc.subcore_barrier()

        base = wid * epw

        def sup_body(sup_i, _):
            sbase = base + sup_i * SUP
            pltpu.sync_copy(dst.at[pl.ds(sbase, SUP)], dst_sup)

            def chunk(cc, __):
                t = sup_i * SUBS + cc
                off = cc * CH
                ebase = sbase + off

                def go(dstb, rows, pb, ssem):
                    @pl.when(t >= 2)
                    def _():
                        pltpu.make_async_copy(
                            pb, den_sh.at[dstb], ssem).wait()
                        pltpu.make_async_copy(
                            rows, num_sh.at[dstb], ssem).wait()
                    for g in range(CH // L):
                        dstb[pl.ds(g * L, L)] = dst_sup[
                            pl.ds(off + g * L, L)]
                    cg = pltpu.async_copy(pg.at[pl.ds(ebase, CH)],
                                          rows, gsem)
                    pltpu.sync_copy(pvec.at[pl.ds(ebase, CH)], pb)
                    cg.wait()
                    pltpu.async_copy(pb, den_sh.at[dstb], ssem,
                                     add=True)
                    pltpu.async_copy(rows, num_sh.at[dstb], ssem,
                                     add=True)

                @pl.when(t % 2 == 0)
                def _():
                    go(dstb2.at[0], rows2.at[0], pb2.at[0], ssem_a)

                @pl.when(t % 2 == 1)
                def _():
                    go(dstb2.at[1], rows2.at[1], pb2.at[1], ssem_b)

                return 0

            return lax.fori_loop(0, SUBS, chunk, 0)

        lax.fori_loop(0, nsup, sup_body, 0)

        for b, ssem in ((0, ssem_a), (1, ssem_b)):
            pltpu.make_async_copy(pb2.at[b], den_sh.at[dstb2.at[b]],
                                  ssem).wait()
            pltpu.make_async_copy(rows2.at[b], num_sh.at[dstb2.at[b]],
                                  ssem).wait()
        plsc.subcore_barrier()

        pltpu.sync_copy(num_sh.at[pl.ds(s * SP0, SP0)],
                        num_out.at[c, pl.ds(s * SP0, SP0)])

        @pl.when(s == NS - 1)
        def _():
            pltpu.sync_copy(num_sh.at[pl.ds(N - 16, 16)],
                            num_out.at[c, pl.ds(N - 16, 16)])

        @pl.when(s == 0)
        def _():
            pltpu.sync_copy(den_sh, den_out.at[c, 0])

    return _es


def _final_body(num0_ref, num1_ref, num2_ref, den_ref, b2_ref, out_ref):
    nsum = (num0_ref[0] + num0_ref[1] + num1_ref[0] + num1_ref[1]
            + num2_ref[0] + num2_ref[1])
    d = jnp.sum(den_ref[...], axis=1)[:, None]
    safe = jnp.where(d > 0.0, d, 1.0)
    out_ref[...] = jnp.where(d > 0.0, nsum / safe, 0.0) + b2_ref[...]


def _final(num0, num1, num2, den6, bias2):
    return pl.pallas_call(
        _final_body,
        grid=(NBLK,),
        in_specs=[
            pl.BlockSpec((NC, BLK, D), lambda i: (0, i, 0)),
            pl.BlockSpec((NC, BLK, D), lambda i: (0, i, 0)),
            pl.BlockSpec((NC, BLK, D), lambda i: (0, i, 0)),
            pl.BlockSpec((BLK, 3 * NC), lambda i: (i, 0)),
            pl.BlockSpec((1, D), lambda i: (0, 0)),
        ],
        out_specs=pl.BlockSpec((BLK, D), lambda i: (i, 0)),
        out_shape=jax.ShapeDtypeStruct((N, D), jnp.float32),
    )(num0, num1, num2, den6, bias2)


E0 = 128000
E1 = 128000
E2 = E - E0 - E1
_eg0 = _make_egather(E0)
_eg1 = _make_egather(E1)
_eg2 = _make_egather(E2)
_es0 = _make_escatter(E0)
_es1 = _make_escatter(E1)
_es2 = _make_escatter(E2)


def kernel(node_features, edge_index, edge_norm, edge_type, comp, bases,
           root, bias1, lin_l, lin_r, att, bias2):
    del edge_norm
    src = edge_index[0]
    dst = edge_index[1]

    comp_pad = jnp.concatenate([comp, jnp.zeros((1, NB), comp.dtype)], axis=0)
    xw_all = _proj(comp_pad, bases, root, node_features)
    table = xw_all[:R].reshape(R * N, D)
    xroot = xw_all[R]

    agg2, cnt32 = _rgcn(table, src, dst, edge_type)

    xl, xr, mm = _post(agg2, cnt32.reshape(NW, N).T, xroot,
                       bias1.reshape(1, D), lin_l, lin_r)
    mb = _bound(mm, xr, att.reshape(1, D))[:, 0]

    src0, dst0 = src[:E0], dst[:E0]
    src1, dst1 = src[E0:E0 + E1], dst[E0:E0 + E1]
    src2, dst2 = src[E0 + E1:], dst[E0 + E1:]
    att2 = att.reshape(D, 1)

    gxl0, gxr0, gmb0 = _eg0(xl, xr, mb, src0, dst0)
    gxl1, gxr1, gmb1 = _eg1(xl, xr, mb, src1, dst1)
    gxl2, gxr2, gmb2v = _eg2(xl, xr, mb, src2, dst2)
    pg0, p0 = _pw(gxl0, gxr0, gmb0.reshape(E0, 1), att2)
    pg1, p1 = _pw(gxl1, gxr1, gmb1.reshape(E1, 1), att2)
    pg2, p2 = _pw(gxl2, gxr2, gmb2v.reshape(E2, 1), att2)
    num0, den0 = _es0(pg0, p0.reshape(E0), dst0)
    num1, den1 = _es1(pg1, p1.reshape(E1), dst1)
    num2, den2 = _es2(pg2, p2.reshape(E2), dst2)

    den6 = jnp.concatenate([den0.reshape(NC, N), den1.reshape(NC, N),
                            den2.reshape(NC, N)], 0).T
    return _final(num0, num1, num2, den6, bias2.reshape(1, D))


# speedup vs baseline: 7.1514x; 1.0393x over previous
import functools
import jax
import jax.numpy as jnp
from jax import lax
from jax.experimental import pallas as pl
from jax.experimental.pallas import tpu as pltpu
from jax.experimental.pallas import tpu_sc as plsc

N, E, D = 10000, 320000, 128
R, NB = 8, 30
NC, NS, L = 2, 16, 16
NW = NC * NS
EPW = E // NW
CH = 80
SUP = 2000
NSUP = EPW // SUP
SUBS = SUP // CH
SP0 = 624
BLK = 400
NBLK = N // BLK

_mesh = plsc.VectorSubcoreMesh(
    core_axis_name="c", subcore_axis_name="s", num_cores=NC, num_subcores=NS)


def _proj_body(comp_ref, bases_ref, root_ref, x_ref, out_ref):
    r = pl.program_id(0)
    w = jnp.zeros((D, D), jnp.float32)
    for b in range(NB):
        w = w + comp_ref[r, b] * bases_ref[b]
    isroot = jnp.where(r == R, 1.0, 0.0)
    w = w + isroot * root_ref[...]
    out_ref[0] = jnp.dot(x_ref[...], w, preferred_element_type=jnp.float32)


def _proj(comp_pad, bases, root, x):
    return pl.pallas_call(
        _proj_body,
        grid=(R + 1, NBLK),
        in_specs=[
            pl.BlockSpec(memory_space=pltpu.SMEM),
            pl.BlockSpec((NB, D, D), lambda r, i: (0, 0, 0)),
            pl.BlockSpec((D, D), lambda r, i: (0, 0)),
            pl.BlockSpec((BLK, D), lambda r, i: (i, 0)),
        ],
        out_specs=pl.BlockSpec((1, BLK, D), lambda r, i: (r, i, 0)),
        out_shape=jax.ShapeDtypeStruct((R + 1, N, D), jnp.float32),
    )(comp_pad, bases, root, x)


def _zero_rows(rows):
    z = jnp.zeros((L,), jnp.float32)

    def body(j, _):
        def inner(k, __):
            rows[j, pl.ds(k * L, L)] = z
            return 0
        return lax.fori_loop(0, D // L, inner, 0)
    lax.fori_loop(0, CH, body, 0)


def _zero_spmem(rows, sh, s):
    for k in range(7):
        pltpu.sync_copy(rows, sh.at[pl.ds(s * SP0 + k * CH, CH)])
    pltpu.sync_copy(rows.at[pl.ds(0, 64)], sh.at[pl.ds(s * SP0 + 7 * CH, 64)])

    @pl.when(s == NS - 1)
    def _():
        pltpu.sync_copy(rows.at[pl.ds(0, 16)], sh.at[pl.ds(N - 16, 16)])


def _zero_1d(buf, n):
    z = jnp.zeros((L,), jnp.float32)

    def body(i, _):
        buf[pl.ds(i * L, L)] = z
        return 0
    lax.fori_loop(0, n // L, body, 0)


@functools.partial(
    pl.kernel,
    out_type=(jax.ShapeDtypeStruct((NC, N, D), jnp.float32),
              jax.ShapeDtypeStruct((NW, 1, N), jnp.float32)),
    mesh=_mesh,
    compiler_params=pltpu.CompilerParams(needs_layout_passes=False),
    scratch_types=[
        pltpu.VMEM((SUP,), jnp.int32),
        pltpu.VMEM((SUP,), jnp.int32),
        pltpu.VMEM((SUP,), jnp.int32),
        pltpu.VMEM((2, CH), jnp.int32),
        pltpu.VMEM((2, CH), jnp.int32),
        pltpu.VMEM((2, CH, D), jnp.float32),
        pltpu.VMEM((N,), jnp.float32),
        pltpu.VMEM_SHARED((N, D), jnp.float32),
        pltpu.SemaphoreType.DMA,
        pltpu.SemaphoreType.DMA,
        pltpu.SemaphoreType.DMA,
    ],
)
def _rgcn(table, src, dst, typ, agg_out, cnt_out,
          src_sup, dst_sup, typ_sup, flatb2, dstb2, rows2, cnt_local,
          agg_sh, gsem, ssem_a, ssem_b):
    c = lax.axis_index("c")
    s = lax.axis_index("s")
    wid = s * NC + c

    _zero_rows(rows2.at[0])
    _zero_spmem(rows2.at[0], agg_sh, s)
    _zero_rows(rows2.at[1])
    _zero_1d(cnt_local, N)
    plsc.subcore_barrier()

    base = wid * EPW
    ones = jnp.ones((L,), jnp.float32)

    def sup_body(sup_i, _):
        sbase = base + sup_i * SUP
        pltpu.sync_copy(src.at[pl.ds(sbase, SUP)], src_sup)
        pltpu.sync_copy(dst.at[pl.ds(sbase, SUP)], dst_sup)
        pltpu.sync_copy(typ.at[pl.ds(sbase, SUP)], typ_sup)

        def chunk(cc, __):
            t = sup_i * SUBS + cc
            off = cc * CH

            def go(flatb, dstb, rows, ssem):
                @pl.when(t >= 2)
                def _():
                    pltpu.make_async_copy(
                        rows, agg_sh.at[dstb], ssem).wait()
                for g in range(CH // L):
                    sl = pl.ds(off + g * L, L)
                    s16 = src_sup[sl]
                    t16 = typ_sup[sl]
                    d16 = dst_sup[sl]
                    flatb[pl.ds(g * L, L)] = t16 * N + s16
                    dstb[pl.ds(g * L, L)] = d16
                    plsc.addupdate_scatter(cnt_local, [d16], ones)
                pltpu.async_copy(table.at[flatb], rows, gsem).wait()
                pltpu.async_copy(rows, agg_sh.at[dstb], ssem, add=True)

            @pl.when(t % 2 == 0)
            def _():
                go(flatb2.at[0], dstb2.at[0], rows2.at[0], ssem_a)

            @pl.when(t % 2 == 1)
            def _():
                go(flatb2.at[1], dstb2.at[1], rows2.at[1], ssem_b)

            return 0

        return lax.fori_loop(0, SUBS, chunk, 0)

    lax.fori_loop(0, NSUP, sup_body, 0)

    for b, ssem in ((0, ssem_a), (1, ssem_b)):
        pltpu.make_async_copy(rows2.at[b], agg_sh.at[dstb2.at[b]],
                              ssem).wait()
    plsc.subcore_barrier()

    pltpu.sync_copy(agg_sh.at[pl.ds(s * SP0, SP0)],
                    agg_out.at[c, pl.ds(s * SP0, SP0)])

    @pl.when(s == NS - 1)
    def _():
        pltpu.sync_copy(agg_sh.at[pl.ds(N - 16, 16)],
                        agg_out.at[c, pl.ds(N - 16, 16)])

    pltpu.sync_copy(cnt_local, cnt_out.at[wid, 0])


def _post_body(agg_ref, cnt_ref, xroot_ref, b1_ref, ll_ref, lr_ref,
               xl_ref, xr_ref, mm_ref):
    aggsum = agg_ref[0] + agg_ref[1]
    cnt = jnp.sum(cnt_ref[...], axis=1)
    x1 = (aggsum / jnp.clip(cnt, 1.0)[:, None]
          + xroot_ref[...] + b1_ref[...])
    xl = jnp.dot(x1, ll_ref[...], preferred_element_type=jnp.float32)
    xr = jnp.dot(x1, lr_ref[...], preferred_element_type=jnp.float32)
    xl_ref[...] = xl
    xr_ref[...] = xr
    mm_ref[0, 0] = jnp.max(xl, axis=0)
    mm_ref[0, 1] = jnp.min(xl, axis=0)


def _post(agg2, cnt32, xroot, bias1, lin_l, lin_r):
    return pl.pallas_call(
        _post_body,
        grid=(NBLK,),
        in_specs=[
            pl.BlockSpec((NC, BLK, D), lambda i: (0, i, 0)),
            pl.BlockSpec((BLK, NW), lambda i: (i, 0)),
            pl.BlockSpec((BLK, D), lambda i: (i, 0)),
            pl.BlockSpec((1, D), lambda i: (0, 0)),
            pl.BlockSpec((D, D), lambda i: (0, 0)),
            pl.BlockSpec((D, D), lambda i: (0, 0)),
        ],
        out_specs=[
            pl.BlockSpec((BLK, D), lambda i: (i, 0)),
            pl.BlockSpec((BLK, D), lambda i: (i, 0)),
            pl.BlockSpec((1, 2, D), lambda i: (i, 0, 0)),
        ],
        out_shape=[
            jax.ShapeDtypeStruct((N, D), jnp.float32),
            jax.ShapeDtypeStruct((N, D), jnp.float32),
            jax.ShapeDtypeStruct((NBLK, 2, D), jnp.float32),
        ],
    )(agg2, cnt32, xroot, bias1, lin_l, lin_r)


def _bound_body(mm_ref, xr_ref, att_ref, mb_ref):
    xlmax = jnp.max(mm_ref[:, 0, :], axis=0)
    xlmin = jnp.min(mm_ref[:, 1, :], axis=0)
    attv = att_ref[0]
    ap = jnp.maximum(attv, 0.0)
    an = jnp.minimum(attv, 0.0)
    zp = xlmax[None, :] + xr_ref[...]
    zn = xlmin[None, :] + xr_ref[...]
    lrp = jnp.maximum(zp, 0.2 * zp)
    lrn = jnp.maximum(zn, 0.2 * zn)
    mb_ref[...] = jnp.sum(ap * lrp + an * lrn, axis=1, keepdims=True)


def _bound(mm, xr, att):
    return pl.pallas_call(
        _bound_body,
        grid=(NBLK,),
        in_specs=[
            pl.BlockSpec((NBLK, 2, D), lambda i: (0, 0, 0)),
            pl.BlockSpec((BLK, D), lambda i: (i, 0)),
            pl.BlockSpec((1, D), lambda i: (0, 0)),
        ],
        out_specs=pl.BlockSpec((BLK, 1), lambda i: (i, 0)),
        out_shape=jax.ShapeDtypeStruct((N, 1), jnp.float32),
    )(mm, xr, att)


def _make_egather(eh):
    epw = eh // NW
    nsup = epw // SUP

    @functools.partial(
        pl.kernel,
        out_type=(jax.ShapeDtypeStruct((eh, D), jnp.float32),
                  jax.ShapeDtypeStruct((eh, D), jnp.float32),
                  jax.ShapeDtypeStruct((eh,), jnp.float32)),
        mesh=_mesh,
        compiler_params=pltpu.CompilerParams(needs_layout_passes=False),
        scratch_types=[
            pltpu.VMEM((SUP,), jnp.int32),
            pltpu.VMEM((SUP,), jnp.int32),
            pltpu.VMEM((CH,), jnp.int32),
            pltpu.VMEM((CH,), jnp.int32),
            pltpu.VMEM((2, CH, D), jnp.float32),
            pltpu.VMEM((2, CH, D), jnp.float32),
            pltpu.VMEM((2, CH), jnp.float32),
            pltpu.VMEM((N,), jnp.float32),
            pltpu.SemaphoreType.DMA,
            pltpu.SemaphoreType.DMA,
            pltpu.SemaphoreType.DMA,
            pltpu.SemaphoreType.DMA,
        ],
    )
    def _eg(xl, xr, mb, src, dst, gxl_out, gxr_out, gmb_out,
            src_sup, dst_sup, srcb, dstb, rows_l2, rows_r2, gmbb2,
            mb_local, lsem, rsem, wsem_a, wsem_b):
        c = lax.axis_index("c")
        s = lax.axis_index("s")
        wid = s * NC + c

        pltpu.sync_copy(mb, mb_local)

        base = wid * epw

        def sup_body(sup_i, _):
            sbase = base + sup_i * SUP
            pltpu.sync_copy(src.at[pl.ds(sbase, SUP)], src_sup)
            pltpu.sync_copy(dst.at[pl.ds(sbase, SUP)], dst_sup)

            def chunk(cc, __):
                t = sup_i * SUBS + cc
                off = cc * CH
                for g in range(CH // L):
                    sl = pl.ds(off + g * L, L)
                    srcb[pl.ds(g * L, L)] = src_sup[sl]
                    dstb[pl.ds(g * L, L)] = dst_sup[sl]
                ebase = sbase + off
                dl = gxl_out.at[pl.ds(ebase, CH)]
                dr = gxr_out.at[pl.ds(ebase, CH)]
                dm = gmb_out.at[pl.ds(ebase, CH)]

                def go(rows_l, rows_r, gmbb, wsem):
                    @pl.when(t >= 2)
                    def _():
                        pltpu.make_async_copy(rows_l, dl, wsem).wait()
                        pltpu.make_async_copy(rows_r, dr, wsem).wait()
                        pltpu.make_async_copy(gmbb, dm, wsem).wait()
                    cl = pltpu.async_copy(xl.at[srcb], rows_l, lsem)
                    cr = pltpu.async_copy(xr.at[dstb], rows_r, rsem)
                    for g in range(CH // L):
                        d16 = dstb[pl.ds(g * L, L)]
                        gmbb[pl.ds(g * L, L)] = plsc.load_gather(
                            mb_local, [d16])
                    cl.wait()
                    cr.wait()
                    pltpu.async_copy(rows_l, dl, wsem)
                    pltpu.async_copy(rows_r, dr, wsem)
                    pltpu.async_copy(gmbb, dm, wsem)

                @pl.when(t % 2 == 0)
                def _():
                    go(rows_l2.at[0], rows_r2.at[0], gmbb2.at[0], wsem_a)

                @pl.when(t % 2 == 1)
                def _():
                    go(rows_l2.at[1], rows_r2.at[1], gmbb2.at[1], wsem_b)

                return 0

            return lax.fori_loop(0, SUBS, chunk, 0)

        lax.fori_loop(0, nsup, sup_body, 0)

        dl = gxl_out.at[pl.ds(base, CH)]
        dr = gxr_out.at[pl.ds(base, CH)]
        dm = gmb_out.at[pl.ds(base, CH)]
        for b, wsem in ((0, wsem_a), (1, wsem_b)):
            pltpu.make_async_copy(rows_l2.at[b], dl, wsem).wait()
            pltpu.make_async_copy(rows_r2.at[b], dr, wsem).wait()
            pltpu.make_async_copy(gmbb2.at[b], dm, wsem).wait()

    return _eg


BE = 6400


def _pw_body(gxl_ref, gxr_ref, gmb_ref, att_ref, pg_ref, p_ref):
    gxl = gxl_ref[...]
    t = gxl + gxr_ref[...]
    lr = jnp.maximum(t, 0.2 * t)
    score = jnp.dot(lr, att_ref[...], preferred_element_type=jnp.float32)
    p = jnp.exp(score - gmb_ref[...])
    pg_ref[...] = p * gxl
    p_ref[...] = p


def _pw(gxl, gxr, gmb2, att2):
    eh = gxl.shape[0]
    return pl.pallas_call(
        _pw_body,
        grid=(eh // BE,),
        in_specs=[
            pl.BlockSpec((BE, D), lambda i: (i, 0)),
            pl.BlockSpec((BE, D), lambda i: (i, 0)),
            pl.BlockSpec((BE, 1), lambda i: (i, 0)),
            pl.BlockSpec((D, 1), lambda i: (0, 0)),
        ],
        out_specs=[
            pl.BlockSpec((BE, D), lambda i: (i, 0)),
            pl.BlockSpec((BE, 1), lambda i: (i, 0)),
        ],
        out_shape=[
            jax.ShapeDtypeStruct((eh, D), jnp.float32),
            jax.ShapeDtypeStruct((eh, 1), jnp.float32),
        ],
    )(gxl, gxr, gmb2, att2)


def _make_escatter(eh):
    epw = eh // NW
    nsup = epw // SUP

    @functools.partial(
        pl.kernel,
        out_type=(jax.ShapeDtypeStruct((NC, N, D), jnp.float32),
                  jax.ShapeDtypeStruct((NC, 1, N), jnp.float32)),
        mesh=_mesh,
        compiler_params=pltpu.CompilerParams(needs_layout_passes=False),
        scratch_types=[
            pltpu.VMEM((SUP,), jnp.int32),
            pltpu.VMEM((2, CH), jnp.int32),
            pltpu.VMEM((2, CH, D), jnp.float32),
            pltpu.VMEM((2, CH), jnp.float32),
            pltpu.VMEM((N,), jnp.float32),
            pltpu.VMEM_SHARED((N, D), jnp.float32),
            pltpu.VMEM_SHARED((N,), jnp.float32),
            pltpu.SemaphoreType.DMA,
            pltpu.SemaphoreType.DMA,
            pltpu.SemaphoreType.DMA,
        ],
    )
    def _es(pg, pvec, dst, num_out, den_out,
            dst_sup, dstb2, rows2, pb2, zbuf, num_sh, den_sh,
            gsem, ssem_a, ssem_b):
        c = lax.axis_index("c")
        s = lax.axis_index("s")
        wid = s * NC + c

        _zero_rows(rows2.at[0])
        _zero_spmem(rows2.at[0], num_sh, s)
        _zero_rows(rows2.at[1])
        _zero_1d(zbuf, N)

        @pl.when(s == 0)
        def _():
            pltpu.sync_copy(zbuf, den_sh)

        plsc.subcore_barrier()

        base = wid * epw

        def sup_body(sup_i, _):
            sbase = base + sup_i * SUP
            pltpu.sync_copy(dst.at[pl.ds(sbase, SUP)], dst_sup)

            def chunk(cc, __):
                t = sup_i * SUBS + cc
                off = cc * CH
                ebase = sbase + off

                def go(dstb, rows, pb, ssem):
                    @pl.when(t >= 2)
                    def _():
                        pltpu.make_async_copy(
                            pb, den_sh.at[dstb], ssem).wait()
                        pltpu.make_async_copy(
                            rows, num_sh.at[dstb], ssem).wait()
                    for g in range(CH // L):
                        dstb[pl.ds(g * L, L)] = dst_sup[
                            pl.ds(off + g * L, L)]
                    cg = pltpu.async_copy(pg.at[pl.ds(ebase, CH)],
                                          rows, gsem)
                    pltpu.sync_copy(pvec.at[pl.ds(ebase, CH)], pb)
                    cg.wait()
                    pltpu.async_copy(pb, den_sh.at[dstb], ssem,
                                     add=True)
                    pltpu.async_copy(rows, num_sh.at[dstb], ssem,
                                     add=True)

                @pl.when(t % 2 == 0)
                def _():
                    go(dstb2.at[0], rows2.at[0], pb2.at[0], ssem_a)

                @pl.when(t % 2 == 1)
                def _():
                    go(dstb2.at[1], rows2.at[1], pb2.at[1], ssem_b)

                return 0

            return lax.fori_loop(0, SUBS, chunk, 0)

        lax.fori_loop(0, nsup, sup_body, 0)

        for b, ssem in ((0, ssem_a), (1, ssem_b)):
            pltpu.make_async_copy(pb2.at[b], den_sh.at[dstb2.at[b]],
                                  ssem).wait()
            pltpu.make_async_copy(rows2.at[b], num_sh.at[dstb2.at[b]],
                                  ssem).wait()
        plsc.subcore_barrier()

        pltpu.sync_copy(num_sh.at[pl.ds(s * SP0, SP0)],
                        num_out.at[c, pl.ds(s * SP0, SP0)])

        @pl.when(s == NS - 1)
        def _():
            pltpu.sync_copy(num_sh.at[pl.ds(N - 16, 16)],
                            num_out.at[c, pl.ds(N - 16, 16)])

        @pl.when(s == 0)
        def _():
            pltpu.sync_copy(den_sh, den_out.at[c, 0])

    return _es


def _final_body(num0_ref, num1_ref, num2_ref, den_ref, b2_ref, out_ref):
    nsum = (num0_ref[0] + num0_ref[1] + num1_ref[0] + num1_ref[1]
            + num2_ref[0] + num2_ref[1])
    d = jnp.sum(den_ref[...], axis=1)[:, None]
    safe = jnp.where(d > 0.0, d, 1.0)
    out_ref[...] = jnp.where(d > 0.0, nsum / safe, 0.0) + b2_ref[...]


def _final(num0, num1, num2, den6, bias2):
    return pl.pallas_call(
        _final_body,
        grid=(NBLK,),
        in_specs=[
            pl.BlockSpec((NC, BLK, D), lambda i: (0, i, 0)),
            pl.BlockSpec((NC, BLK, D), lambda i: (0, i, 0)),
            pl.BlockSpec((NC, BLK, D), lambda i: (0, i, 0)),
            pl.BlockSpec((BLK, 3 * NC), lambda i: (i, 0)),
            pl.BlockSpec((1, D), lambda i: (0, 0)),
        ],
        out_specs=pl.BlockSpec((BLK, D), lambda i: (i, 0)),
        out_shape=jax.ShapeDtypeStruct((N, D), jnp.float32),
    )(num0, num1, num2, den6, bias2)


E0 = 128000
E1 = 128000
E2 = E - E0 - E1
_eg0 = _make_egather(E0)
_eg1 = _make_egather(E1)
_eg2 = _make_egather(E2)
_es0 = _make_escatter(E0)
_es1 = _make_escatter(E1)
_es2 = _make_escatter(E2)


def kernel(node_features, edge_index, edge_norm, edge_type, comp, bases,
           root, bias1, lin_l, lin_r, att, bias2):
    del edge_norm
    src = edge_index[0]
    dst = edge_index[1]

    comp_pad = jnp.concatenate([comp, jnp.zeros((1, NB), comp.dtype)], axis=0)
    xw_all = _proj(comp_pad, bases, root, node_features)
    table = xw_all[:R].reshape(R * N, D)
    xroot = xw_all[R]

    agg2, cnt32 = _rgcn(table, src, dst, edge_type)

    xl, xr, mm = _post(agg2, cnt32.reshape(NW, N).T, xroot,
                       bias1.reshape(1, D), lin_l, lin_r)
    mb = _bound(mm, xr, att.reshape(1, D))[:, 0]

    src0, dst0 = src[:E0], dst[:E0]
    src1, dst1 = src[E0:E0 + E1], dst[E0:E0 + E1]
    src2, dst2 = src[E0 + E1:], dst[E0 + E1:]
    att2 = att.reshape(D, 1)

    gxl0, gxr0, gmb0 = _eg0(xl, xr, mb, src0, dst0)
    gxl1, gxr1, gmb1 = _eg1(xl, xr, mb, src1, dst1)
    gxl2, gxr2, gmb2v = _eg2(xl, xr, mb, src2, dst2)
    pg0, p0 = _pw(gxl0, gxr0, gmb0.reshape(E0, 1), att2)
    pg1, p1 = _pw(gxl1, gxr1, gmb1.reshape(E1, 1), att2)
    pg2, p2 = _pw(gxl2, gxr2, gmb2v.reshape(E2, 1), att2)
    num0, den0 = _es0(pg0, p0.reshape(E0), dst0)
    num1, den1 = _es1(pg1, p1.reshape(E1), dst1)
    num2, den2 = _es2(pg2, p2.reshape(E2), dst2)

    den6 = jnp.concatenate([den0.reshape(NC, N), den1.reshape(NC, N),
                            den2.reshape(NC, N)], 0).T
    return _final(num0, num1, num2, den6, bias2.reshape(1, D))
